# baseline (speedup 1.0000x reference)
import functools
import jax
import jax.numpy as jnp
from jax import lax
from jax.experimental import pallas as pl
from jax.experimental.pallas import tpu as pltpu
from jax.experimental.pallas import tpu_sc as plsc

N = 10000
D = 128
E = 320000
NC = 2
NS = 16
NW = NC * NS
EPT = E // NW
CH = 80
NCHUNK = EPT // CH
CHD = 80
NCHD = EPT // CHD
ROWB = 1000
ZB = 40

_f32 = jnp.float32

_sc_mesh = plsc.VectorSubcoreMesh(core_axis_name="c", subcore_axis_name="s")


@functools.partial(
    pl.kernel,
    mesh=_sc_mesh,
    out_type=jax.ShapeDtypeStruct((NC * N,), _f32),
    scratch_types=[
        pltpu.VMEM((NCHD, CHD), jnp.int32),
        pltpu.VMEM((CHD,), _f32),
        pltpu.VMEM((ROWB,), _f32),
        pltpu.VMEM_SHARED((N,), _f32),
    ],
)
def _sc_deg(edges_hbm, ones_hbm, zeros_hbm, out_hbm, rowv, onesv, stg, degs):
    cid = lax.axis_index("c")
    sid = lax.axis_index("s")
    wid = cid * NS + sid

    pltpu.sync_copy(edges_hbm.at[wid], rowv)
    pltpu.sync_copy(ones_hbm, onesv)

    @pl.when(sid < N // ROWB)
    def _():
        pltpu.sync_copy(zeros_hbm, stg)
        pltpu.sync_copy(stg, degs.at[pl.ds(sid * ROWB, ROWB)])

    plsc.subcore_barrier()

    def body(g, carry):
        pltpu.sync_copy(onesv, degs.at[rowv.at[g]], add=True)
        return carry

    lax.fori_loop(0, NCHD, body, 0, unroll=False)

    plsc.subcore_barrier()

    @pl.when(sid < N // ROWB)
    def _():
        pltpu.sync_copy(degs.at[pl.ds(sid * ROWB, ROWB)], stg)
        pltpu.sync_copy(stg, out_hbm.at[pl.ds(cid * N + sid * ROWB, ROWB)])


@functools.partial(
    pl.kernel,
    mesh=_sc_mesh,
    out_type=jax.ShapeDtypeStruct((NC, N, D), _f32),
    scratch_types=[
        pltpu.VMEM((EPT,), jnp.int32),
        pltpu.VMEM((CH,), jnp.int32),
        pltpu.VMEM((CH,), jnp.int32),
        pltpu.VMEM((CH,), jnp.int32),
        pltpu.VMEM((CH,), jnp.int32),
        pltpu.VMEM((CH, D), _f32),
        pltpu.VMEM((CH, D), _f32),
        pltpu.VMEM_SHARED((N, D), _f32),
        pltpu.SemaphoreType.DMA,
        pltpu.SemaphoreType.DMA,
    ],
)
def _sc_spmm(epk_hbm, y_hbm, zeros_hbm, out_hbm,
             pk, rowca, colca, rowcb, colcb, bufa, bufb, agg, sema, semb):
    cid = lax.axis_index("c")
    sid = lax.axis_index("s")
    wid = cid * NS + sid

    pltpu.sync_copy(epk_hbm.at[wid], pk)

    @pl.when(sid < N // ROWB)
    def _():
        pltpu.sync_copy(zeros_hbm, bufa.at[pl.ds(0, ZB)])

        def zbody(j, carry):
            pltpu.sync_copy(bufa.at[pl.ds(0, ZB)],
                            agg.at[pl.ds(sid * ROWB + j * ZB, ZB)])
            return carry

        lax.fori_loop(0, ROWB // ZB, zbody, 0, unroll=False)

    plsc.subcore_barrier()

    def unpack(g, rowc, colc):
        for k in range(CH // 16):
            v = pk[pl.ds(g * CH + 16 * k, 16)]
            rowc[pl.ds(16 * k, 16)] = lax.shift_right_logical(v, 14)
            colc[pl.ds(16 * k, 16)] = lax.bitwise_and(v, (1 << 14) - 1)

    unpack(0, rowca, colca)
    pltpu.make_async_copy(y_hbm.at[colca], bufa, sema).start()

    def body(t, carry):
        ga = 2 * t
        unpack(ga + 1, rowcb, colcb)
        pltpu.make_async_copy(y_hbm.at[colcb], bufb, semb).start()
        pltpu.make_async_copy(y_hbm.at[colca], bufa, sema).wait()
        pltpu.sync_copy(bufa, agg.at[rowca], add=True)
        unpack(ga + 2, rowca, colca)
        pltpu.make_async_copy(y_hbm.at[colca], bufa, sema).start()
        pltpu.make_async_copy(y_hbm.at[colcb], bufb, semb).wait()
        pltpu.sync_copy(bufb, agg.at[rowcb], add=True)
        return carry

    lax.fori_loop(0, (NCHUNK - 1) // 2, body, 0, unroll=False)

    pltpu.make_async_copy(y_hbm.at[colca], bufa, sema).wait()
    pltpu.sync_copy(bufa, agg.at[rowca], add=True)

    plsc.subcore_barrier()

    @pl.when(sid < N // ROWB)
    def _():
        def obody(j, carry):
            base = sid * ROWB + j * ZB
            pltpu.sync_copy(agg.at[pl.ds(base, ZB)], bufa.at[pl.ds(0, ZB)])
            pltpu.sync_copy(bufa.at[pl.ds(0, ZB)],
                            out_hbm.at[cid, pl.ds(base, ZB)])
            return carry

        lax.fori_loop(0, ROWB // ZB, obody, 0, unroll=False)


RB = 1000
GRID = N // RB


def _tc_prep_body(degp_ref, x_ref, w_ref, b_ref, dis_ref, h_ref, y_ref):
    deg = degp_ref[:, 0:1] + degp_ref[:, 1:2]
    dis = jnp.where(deg > 0.0,
                    lax.rsqrt(jnp.maximum(deg, 1e-12)), 0.0)
    h = lax.dot_general(x_ref[...], w_ref[...],
                        (((1,), (1,)), ((), ())),
                        preferred_element_type=_f32) + b_ref[...]
    dis_ref[...] = dis
    h_ref[...] = h
    y_ref[...] = h * dis


def _tc_prep(degp, x, w_in, b_in):
    return pl.pallas_call(
        _tc_prep_body,
        grid=(GRID,),
        in_specs=[
            pl.BlockSpec((RB, NC), lambda b: (b, 0)),
            pl.BlockSpec((RB, D), lambda b: (b, 0)),
            pl.BlockSpec((D, D), lambda b: (0, 0)),
            pl.BlockSpec((1, D), lambda b: (0, 0)),
        ],
        out_specs=[
            pl.BlockSpec((RB, 1), lambda b: (b, 0)),
            pl.BlockSpec((RB, D), lambda b: (b, 0)),
            pl.BlockSpec((RB, D), lambda b: (b, 0)),
        ],
        out_shape=[
            jax.ShapeDtypeStruct((N, 1), _f32),
            jax.ShapeDtypeStruct((N, D), _f32),
            jax.ShapeDtypeStruct((N, D), _f32),
        ],
    )(degp, x, w_in, b_in)


def _bn_from_acc(acc_ref):
    mean = acc_ref[0:1, :] / float(N)
    var = acc_ref[1:2, :] / float(N) - mean * mean
    return mean, lax.rsqrt(var + 1e-5)


def _layer_phase1(bm, part_ref, dis_ref, w_ref, b_ref, tbuf_ref, acc_ref):
    b = pl.program_id(0)
    a = (part_ref[0] + part_ref[1]) * dis_ref[...]
    t = lax.dot_general(a, w_ref[...], (((1,), (1,)), ((), ())),
                        preferred_element_type=_f32) + b_ref[...]
    tbuf_ref[bm] = t

    @pl.when(b == 0)
    def _():
        acc_ref[...] = jnp.zeros_like(acc_ref)

    acc_ref[0:1, :] += jnp.sum(t, axis=0, keepdims=True)
    acc_ref[1:2, :] += jnp.sum(t * t, axis=0, keepdims=True)


def _tc_layer_body(part_ref, dis_ref, w_ref, b_ref, h_ref, g_ref, be_ref,
                   hn_ref, y_ref, tbuf_ref, acc_ref):
    b = pl.program_id(0)
    bm = lax.rem(b, GRID)

    @pl.when(b < GRID)
    def _():
        _layer_phase1(bm, part_ref, dis_ref, w_ref, b_ref, tbuf_ref, acc_ref)

    @pl.when(b >= GRID)
    def _():
        mean, inv = _bn_from_acc(acc_ref)
        tn = (tbuf_ref[bm] - mean) * inv * g_ref[...] + be_ref[...]
        hn = jnp.maximum(tn, 0.0) + h_ref[...]
        hn_ref[...] = hn
        y_ref[...] = hn * dis_ref[...]


def _tc_layer(partial, dis, w, bvec, h, gamma, beta):
    return pl.pallas_call(
        _tc_layer_body,
        grid=(2 * GRID,),
        in_specs=[
            pl.BlockSpec((NC, RB, D), lambda b: (0, lax.rem(b, GRID), 0)),
            pl.BlockSpec((RB, 1), lambda b: (lax.rem(b, GRID), 0)),
            pl.BlockSpec((D, D), lambda b: (0, 0)),
            pl.BlockSpec((1, D), lambda b: (0, 0)),
            pl.BlockSpec((RB, D), lambda b: (lax.rem(b, GRID), 0)),
            pl.BlockSpec((1, D), lambda b: (0, 0)),
            pl.BlockSpec((1, D), lambda b: (0, 0)),
        ],
        out_specs=[
            pl.BlockSpec((RB, D), lambda b: (lax.rem(b, GRID), 0)),
            pl.BlockSpec((RB, D), lambda b: (lax.rem(b, GRID), 0)),
        ],
        out_shape=[
            jax.ShapeDtypeStruct((N, D), _f32),
            jax.ShapeDtypeStruct((N, D), _f32),
        ],
        scratch_shapes=[
            pltpu.VMEM((GRID, RB, D), _f32),
            pltpu.VMEM((2, D), _f32),
        ],
    )(partial, dis, w, bvec, h, gamma, beta)


def _tc_layer_out_body(part_ref, dis_ref, w_ref, b_ref, h_ref, g_ref, be_ref,
                       wo_ref, bo_ref, out_ref, tbuf_ref, acc_ref):
    b = pl.program_id(0)
    bm = lax.rem(b, GRID)

    @pl.when(b < GRID)
    def _():
        _layer_phase1(bm, part_ref, dis_ref, w_ref, b_ref, tbuf_ref, acc_ref)

    @pl.when(b >= GRID)
    def _():
        mean, inv = _bn_from_acc(acc_ref)
        tn = (tbuf_ref[bm] - mean) * inv * g_ref[...] + be_ref[...]
        hn = jnp.maximum(tn, 0.0) + h_ref[...]
        out_ref[...] = lax.dot_general(
            hn, wo_ref[...], (((1,), (1,)), ((), ())),
            preferred_element_type=_f32) + bo_ref[...]


def _tc_layer_out(partial, dis, w, bvec, h, gamma, beta, w_out, b_out):
    return pl.pallas_call(
        _tc_layer_out_body,
        grid=(2 * GRID,),
        in_specs=[
            pl.BlockSpec((NC, RB, D), lambda b: (0, lax.rem(b, GRID), 0)),
            pl.BlockSpec((RB, 1), lambda b: (lax.rem(b, GRID), 0)),
            pl.BlockSpec((D, D), lambda b: (0, 0)),
            pl.BlockSpec((1, D), lambda b: (0, 0)),
            pl.BlockSpec((RB, D), lambda b: (lax.rem(b, GRID), 0)),
            pl.BlockSpec((1, D), lambda b: (0, 0)),
            pl.BlockSpec((1, D), lambda b: (0, 0)),
            pl.BlockSpec((D, D), lambda b: (0, 0)),
            pl.BlockSpec((1, D), lambda b: (0, 0)),
        ],
        out_specs=pl.BlockSpec((RB, D), lambda b: (lax.rem(b, GRID), 0)),
        out_shape=jax.ShapeDtypeStruct((N, D), _f32),
        scratch_shapes=[
            pltpu.VMEM((GRID, RB, D), _f32),
            pltpu.VMEM((2, D), _f32),
        ],
    )(partial, dis, w, bvec, h, gamma, beta, w_out, b_out)


def kernel(x, edge_index, W_in, b_in, W1, b1, gamma1, beta1,
           W2, b2, gamma2, beta2, W_out, b_out):
    ei = edge_index.astype(jnp.int32)
    rows = ei[0].reshape(NW, NCHD, CHD)
    epacked = (ei[0] * (1 << 14) + ei[1]).reshape(NW, EPT)
    ones_e = jnp.ones((CHD,), _f32)
    zeros_n = jnp.zeros((ROWB,), _f32)
    zeros_zd = jnp.zeros((ZB, D), _f32)

    degp = _sc_deg(rows, ones_e, zeros_n)
    dis, h, y = _tc_prep(degp.reshape(NC, N).T, x, W_in, b_in.reshape(1, D))

    part = _sc_spmm(epacked, y, zeros_zd)
    h, y = _tc_layer(part, dis, W1, b1.reshape(1, D), h,
                     gamma1.reshape(1, D), beta1.reshape(1, D))

    part = _sc_spmm(epacked, y, zeros_zd)
    out = _tc_layer_out(part, dis, W2, b2.reshape(1, D), h,
                        gamma2.reshape(1, D), beta2.reshape(1, D),
                        W_out, b_out.reshape(1, D))
    return out


# speedup vs baseline: 1.0381x; 1.0381x over previous
import functools
import jax
import jax.numpy as jnp
from jax import lax
from jax.experimental import pallas as pl
from jax.experimental.pallas import tpu as pltpu
from jax.experimental.pallas import tpu_sc as plsc

N = 10000
D = 128
E = 320000
NC = 2
NS = 16
NW = NC * NS
EPT = E // NW
CH = 80
NCHUNK = EPT // CH
CHD = 80
NCHD = EPT // CHD
ROWB = 1000
ZB = 40

_f32 = jnp.float32

_sc_mesh = plsc.VectorSubcoreMesh(core_axis_name="c", subcore_axis_name="s")


@functools.partial(
    pl.kernel,
    mesh=_sc_mesh,
    out_type=jax.ShapeDtypeStruct((NC * N,), _f32),
    scratch_types=[
        pltpu.VMEM((NCHD, CHD), jnp.int32),
        pltpu.VMEM((CHD,), _f32),
        pltpu.VMEM((ROWB,), _f32),
        pltpu.VMEM_SHARED((N,), _f32),
    ],
)
def _sc_deg(edges_hbm, ones_hbm, zeros_hbm, out_hbm, rowv, onesv, stg, degs):
    cid = lax.axis_index("c")
    sid = lax.axis_index("s")
    wid = cid * NS + sid

    pltpu.sync_copy(edges_hbm.at[wid], rowv)
    pltpu.sync_copy(ones_hbm, onesv)

    @pl.when(sid < N // ROWB)
    def _():
        pltpu.sync_copy(zeros_hbm, stg)
        pltpu.sync_copy(stg, degs.at[pl.ds(sid * ROWB, ROWB)])

    plsc.subcore_barrier()

    def body(g, carry):
        pltpu.sync_copy(onesv, degs.at[rowv.at[g]], add=True)
        return carry

    lax.fori_loop(0, NCHD, body, 0, unroll=False)

    plsc.subcore_barrier()

    @pl.when(sid < N // ROWB)
    def _():
        pltpu.sync_copy(degs.at[pl.ds(sid * ROWB, ROWB)], stg)
        pltpu.sync_copy(stg, out_hbm.at[pl.ds(cid * N + sid * ROWB, ROWB)])


@functools.partial(
    pl.kernel,
    mesh=_sc_mesh,
    out_type=jax.ShapeDtypeStruct((NC, N, D), _f32),
    scratch_types=[
        pltpu.VMEM((EPT,), jnp.int32),
        pltpu.VMEM((CH,), jnp.int32),
        pltpu.VMEM((CH,), jnp.int32),
        pltpu.VMEM((CH,), jnp.int32),
        pltpu.VMEM((CH,), jnp.int32),
        pltpu.VMEM((CH, D), _f32),
        pltpu.VMEM((CH, D), _f32),
        pltpu.VMEM_SHARED((N, D), _f32),
        pltpu.SemaphoreType.DMA,
        pltpu.SemaphoreType.DMA,
    ],
)
def _sc_spmm(epk_hbm, y_hbm, zeros_hbm, out_hbm,
             pk, rowca, colca, rowcb, colcb, bufa, bufb, agg, sema, semb):
    cid = lax.axis_index("c")
    sid = lax.axis_index("s")
    wid = cid * NS + sid

    pltpu.sync_copy(epk_hbm.at[wid], pk)

    @pl.when(sid < N // ROWB)
    def _():
        pltpu.sync_copy(zeros_hbm, agg.at[pl.ds(sid * ROWB, ROWB)])

    plsc.subcore_barrier()

    def unpack(g, rowc, colc):
        for k in range(CH // 16):
            v = pk[pl.ds(g * CH + 16 * k, 16)]
            rowc[pl.ds(16 * k, 16)] = lax.shift_right_logical(v, 14)
            colc[pl.ds(16 * k, 16)] = lax.bitwise_and(v, (1 << 14) - 1)

    unpack(0, rowca, colca)
    pltpu.make_async_copy(y_hbm.at[colca], bufa, sema).start()

    def body(t, carry):
        ga = 2 * t
        unpack(ga + 1, rowcb, colcb)
        pltpu.make_async_copy(y_hbm.at[colcb], bufb, semb).start()
        pltpu.make_async_copy(y_hbm.at[colca], bufa, sema).wait()
        pltpu.sync_copy(bufa, agg.at[rowca], add=True)
        unpack(ga + 2, rowca, colca)
        pltpu.make_async_copy(y_hbm.at[colca], bufa, sema).start()
        pltpu.make_async_copy(y_hbm.at[colcb], bufb, semb).wait()
        pltpu.sync_copy(bufb, agg.at[rowcb], add=True)
        return carry

    lax.fori_loop(0, (NCHUNK - 1) // 2, body, 0, unroll=False)

    pltpu.make_async_copy(y_hbm.at[colca], bufa, sema).wait()
    pltpu.sync_copy(bufa, agg.at[rowca], add=True)

    plsc.subcore_barrier()

    @pl.when(sid < N // ROWB)
    def _():
        pltpu.sync_copy(agg.at[pl.ds(sid * ROWB, ROWB)],
                        out_hbm.at[cid, pl.ds(sid * ROWB, ROWB)])


RB = 1000
GRID = N // RB


def _tc_prep_body(degp_ref, x_ref, w_ref, b_ref, dis_ref, h_ref, y_ref):
    deg = degp_ref[:, 0:1] + degp_ref[:, 1:2]
    dis = jnp.where(deg > 0.0,
                    lax.rsqrt(jnp.maximum(deg, 1e-12)), 0.0)
    h = lax.dot_general(x_ref[...], w_ref[...],
                        (((1,), (1,)), ((), ())),
                        preferred_element_type=_f32) + b_ref[...]
    dis_ref[...] = dis
    h_ref[...] = h
    y_ref[...] = h * dis


def _tc_prep(degp, x, w_in, b_in):
    return pl.pallas_call(
        _tc_prep_body,
        grid=(GRID,),
        in_specs=[
            pl.BlockSpec((RB, NC), lambda b: (b, 0)),
            pl.BlockSpec((RB, D), lambda b: (b, 0)),
            pl.BlockSpec((D, D), lambda b: (0, 0)),
            pl.BlockSpec((1, D), lambda b: (0, 0)),
        ],
        out_specs=[
            pl.BlockSpec((RB, 1), lambda b: (b, 0)),
            pl.BlockSpec((RB, D), lambda b: (b, 0)),
            pl.BlockSpec((RB, D), lambda b: (b, 0)),
        ],
        out_shape=[
            jax.ShapeDtypeStruct((N, 1), _f32),
            jax.ShapeDtypeStruct((N, D), _f32),
            jax.ShapeDtypeStruct((N, D), _f32),
        ],
    )(degp, x, w_in, b_in)


def _bn_from_acc(acc_ref):
    mean = acc_ref[0:1, :] / float(N)
    var = acc_ref[1:2, :] / float(N) - mean * mean
    return mean, lax.rsqrt(var + 1e-5)


def _layer_phase1(bm, part_ref, dis_ref, w_ref, b_ref, tbuf_ref, acc_ref):
    b = pl.program_id(0)
    a = (part_ref[0] + part_ref[1]) * dis_ref[...]
    t = lax.dot_general(a, w_ref[...], (((1,), (1,)), ((), ())),
                        preferred_element_type=_f32) + b_ref[...]
    tbuf_ref[bm] = t

    @pl.when(b == 0)
    def _():
        acc_ref[...] = jnp.zeros_like(acc_ref)

    acc_ref[0:1, :] += jnp.sum(t, axis=0, keepdims=True)
    acc_ref[1:2, :] += jnp.sum(t * t, axis=0, keepdims=True)


def _tc_layer_body(part_ref, dis_ref, w_ref, b_ref, h_ref, g_ref, be_ref,
                   hn_ref, y_ref, tbuf_ref, acc_ref):
    b = pl.program_id(0)
    bm = lax.rem(b, GRID)

    @pl.when(b < GRID)
    def _():
        _layer_phase1(bm, part_ref, dis_ref, w_ref, b_ref, tbuf_ref, acc_ref)

    @pl.when(b >= GRID)
    def _():
        mean, inv = _bn_from_acc(acc_ref)
        tn = (tbuf_ref[bm] - mean) * inv * g_ref[...] + be_ref[...]
        hn = jnp.maximum(tn, 0.0) + h_ref[...]
        hn_ref[...] = hn
        y_ref[...] = hn * dis_ref[...]


def _tc_layer(partial, dis, w, bvec, h, gamma, beta):
    return pl.pallas_call(
        _tc_layer_body,
        grid=(2 * GRID,),
        in_specs=[
            pl.BlockSpec((NC, RB, D), lambda b: (0, lax.rem(b, GRID), 0)),
            pl.BlockSpec((RB, 1), lambda b: (lax.rem(b, GRID), 0)),
            pl.BlockSpec((D, D), lambda b: (0, 0)),
            pl.BlockSpec((1, D), lambda b: (0, 0)),
            pl.BlockSpec((RB, D), lambda b: (lax.rem(b, GRID), 0)),
            pl.BlockSpec((1, D), lambda b: (0, 0)),
            pl.BlockSpec((1, D), lambda b: (0, 0)),
        ],
        out_specs=[
            pl.BlockSpec((RB, D), lambda b: (lax.rem(b, GRID), 0)),
            pl.BlockSpec((RB, D), lambda b: (lax.rem(b, GRID), 0)),
        ],
        out_shape=[
            jax.ShapeDtypeStruct((N, D), _f32),
            jax.ShapeDtypeStruct((N, D), _f32),
        ],
        scratch_shapes=[
            pltpu.VMEM((GRID, RB, D), _f32),
            pltpu.VMEM((2, D), _f32),
        ],
    )(partial, dis, w, bvec, h, gamma, beta)


def _tc_layer_out_body(part_ref, dis_ref, w_ref, b_ref, h_ref, g_ref, be_ref,
                       wo_ref, bo_ref, out_ref, tbuf_ref, acc_ref):
    b = pl.program_id(0)
    bm = lax.rem(b, GRID)

    @pl.when(b < GRID)
    def _():
        _layer_phase1(bm, part_ref, dis_ref, w_ref, b_ref, tbuf_ref, acc_ref)

    @pl.when(b >= GRID)
    def _():
        mean, inv = _bn_from_acc(acc_ref)
        tn = (tbuf_ref[bm] - mean) * inv * g_ref[...] + be_ref[...]
        hn = jnp.maximum(tn, 0.0) + h_ref[...]
        out_ref[...] = lax.dot_general(
            hn, wo_ref[...], (((1,), (1,)), ((), ())),
            preferred_element_type=_f32) + bo_ref[...]


def _tc_layer_out(partial, dis, w, bvec, h, gamma, beta, w_out, b_out):
    return pl.pallas_call(
        _tc_layer_out_body,
        grid=(2 * GRID,),
        in_specs=[
            pl.BlockSpec((NC, RB, D), lambda b: (0, lax.rem(b, GRID), 0)),
            pl.BlockSpec((RB, 1), lambda b: (lax.rem(b, GRID), 0)),
            pl.BlockSpec((D, D), lambda b: (0, 0)),
            pl.BlockSpec((1, D), lambda b: (0, 0)),
            pl.BlockSpec((RB, D), lambda b: (lax.rem(b, GRID), 0)),
            pl.BlockSpec((1, D), lambda b: (0, 0)),
            pl.BlockSpec((1, D), lambda b: (0, 0)),
            pl.BlockSpec((D, D), lambda b: (0, 0)),
            pl.BlockSpec((1, D), lambda b: (0, 0)),
        ],
        out_specs=pl.BlockSpec((RB, D), lambda b: (lax.rem(b, GRID), 0)),
        out_shape=jax.ShapeDtypeStruct((N, D), _f32),
        scratch_shapes=[
            pltpu.VMEM((GRID, RB, D), _f32),
            pltpu.VMEM((2, D), _f32),
        ],
    )(partial, dis, w, bvec, h, gamma, beta, w_out, b_out)


def kernel(x, edge_index, W_in, b_in, W1, b1, gamma1, beta1,
           W2, b2, gamma2, beta2, W_out, b_out):
    ei = edge_index.astype(jnp.int32)
    rows = ei[0].reshape(NW, NCHD, CHD)
    epacked = (ei[0] * (1 << 14) + ei[1]).reshape(NW, EPT)
    ones_e = jnp.ones((CHD,), _f32)
    zeros_n = jnp.zeros((ROWB,), _f32)
    zeros_zd = jnp.zeros((ROWB, D), _f32)

    degp = _sc_deg(rows, ones_e, zeros_n)
    dis, h, y = _tc_prep(degp.reshape(NC, N).T, x, W_in, b_in.reshape(1, D))

    part = _sc_spmm(epacked, y, zeros_zd)
    h, y = _tc_layer(part, dis, W1, b1.reshape(1, D), h,
                     gamma1.reshape(1, D), beta1.reshape(1, D))

    part = _sc_spmm(epacked, y, zeros_zd)
    out = _tc_layer_out(part, dis, W2, b2.reshape(1, D), h,
                        gamma2.reshape(1, D), beta2.reshape(1, D),
                        W_out, b_out.reshape(1, D))
    return out


# speedup vs baseline: 1.0527x; 1.0140x over previous
import functools
import jax
import jax.numpy as jnp
from jax import lax
from jax.experimental import pallas as pl
from jax.experimental.pallas import tpu as pltpu
from jax.experimental.pallas import tpu_sc as plsc

N = 10000
D = 128
E = 320000
NC = 2
NS = 16
NW = NC * NS
EPT = E // NW
CH = 80
NCHUNK = EPT // CH
CHD = 80
NCHD = EPT // CHD
ROWB = 1000
ZB = 40

_f32 = jnp.float32

_sc_mesh = plsc.VectorSubcoreMesh(core_axis_name="c", subcore_axis_name="s")


@functools.partial(
    pl.kernel,
    mesh=_sc_mesh,
    out_type=jax.ShapeDtypeStruct((NC * N,), _f32),
    scratch_types=[
        pltpu.VMEM((NCHD, CHD), jnp.int32),
        pltpu.VMEM((CHD,), _f32),
        pltpu.VMEM((ROWB,), _f32),
        pltpu.VMEM_SHARED((N,), _f32),
    ],
)
def _sc_deg(edges_hbm, ones_hbm, zeros_hbm, out_hbm, rowv, onesv, stg, degs):
    cid = lax.axis_index("c")
    sid = lax.axis_index("s")
    wid = cid * NS + sid

    pltpu.sync_copy(edges_hbm.at[wid], rowv)
    pltpu.sync_copy(ones_hbm, onesv)

    @pl.when(sid < N // ROWB)
    def _():
        pltpu.sync_copy(zeros_hbm, stg)
        pltpu.sync_copy(stg, degs.at[pl.ds(sid * ROWB, ROWB)])

    plsc.subcore_barrier()

    def body(g, carry):
        pltpu.sync_copy(onesv, degs.at[rowv.at[g]], add=True)
        return carry

    lax.fori_loop(0, NCHD, body, 0, unroll=False)

    plsc.subcore_barrier()

    @pl.when(sid < N // ROWB)
    def _():
        pltpu.sync_copy(degs.at[pl.ds(sid * ROWB, ROWB)], stg)
        pltpu.sync_copy(stg, out_hbm.at[pl.ds(cid * N + sid * ROWB, ROWB)])


@functools.partial(
    pl.kernel,
    mesh=_sc_mesh,
    out_type=jax.ShapeDtypeStruct((NC, N, D), _f32),
    scratch_types=[
        pltpu.VMEM((EPT,), jnp.int32),
        pltpu.VMEM((CH,), jnp.int32),
        pltpu.VMEM((CH,), jnp.int32),
        pltpu.VMEM((CH,), jnp.int32),
        pltpu.VMEM((CH,), jnp.int32),
        pltpu.VMEM((CH, D), _f32),
        pltpu.VMEM((CH, D), _f32),
        pltpu.VMEM_SHARED((N, D), _f32),
        pltpu.SemaphoreType.DMA,
        pltpu.SemaphoreType.DMA,
    ],
)
def _sc_spmm(epk_hbm, y_hbm, zeros_hbm, out_hbm,
             pk, rowca, colca, rowcb, colcb, bufa, bufb, agg, sema, semb):
    cid = lax.axis_index("c")
    sid = lax.axis_index("s")
    wid = cid * NS + sid

    pltpu.sync_copy(epk_hbm.at[wid], pk)

    @pl.when(sid < N // ROWB)
    def _():
        pltpu.sync_copy(zeros_hbm, agg.at[pl.ds(sid * ROWB, ROWB)])

    plsc.subcore_barrier()

    def unpack(g, rowc, colc):
        for k in range(CH // 16):
            v = pk[pl.ds(g * CH + 16 * k, 16)]
            rowc[pl.ds(16 * k, 16)] = lax.shift_right_logical(v, 14)
            colc[pl.ds(16 * k, 16)] = lax.bitwise_and(v, (1 << 14) - 1)

    unpack(0, rowca, colca)
    pltpu.make_async_copy(y_hbm.at[colca], bufa, sema).start()

    def body(t, carry):
        ga = 2 * t
        unpack(ga + 1, rowcb, colcb)
        pltpu.make_async_copy(y_hbm.at[colcb], bufb, semb).start()
        pltpu.make_async_copy(y_hbm.at[colca], bufa, sema).wait()
        pltpu.sync_copy(bufa, agg.at[rowca], add=True)
        unpack(ga + 2, rowca, colca)
        pltpu.make_async_copy(y_hbm.at[colca], bufa, sema).start()
        pltpu.make_async_copy(y_hbm.at[colcb], bufb, semb).wait()
        pltpu.sync_copy(bufb, agg.at[rowcb], add=True)
        return carry

    lax.fori_loop(0, (NCHUNK - 1) // 2, body, 0, unroll=False)

    pltpu.make_async_copy(y_hbm.at[colca], bufa, sema).wait()
    pltpu.sync_copy(bufa, agg.at[rowca], add=True)

    plsc.subcore_barrier()

    @pl.when(sid < N // ROWB)
    def _():
        pltpu.sync_copy(agg.at[pl.ds(sid * ROWB, ROWB)],
                        out_hbm.at[cid, pl.ds(sid * ROWB, ROWB)])


RB = 1000
GRID = N // RB


def _tc_prep_body(degp_ref, x_ref, w_ref, b_ref, dis_ref, h_ref, y_ref):
    deg = degp_ref[:, 0:1] + degp_ref[:, 1:2]
    dis = jnp.where(deg > 0.0,
                    lax.rsqrt(jnp.maximum(deg, 1e-12)), 0.0)
    h = lax.dot_general(x_ref[...], w_ref[...],
                        (((1,), (1,)), ((), ())),
                        preferred_element_type=_f32) + b_ref[...]
    dis_ref[...] = dis
    h_ref[...] = h
    y_ref[...] = h * dis


def _tc_prep(degp, x, w_in, b_in):
    return pl.pallas_call(
        _tc_prep_body,
        grid=(GRID,),
        in_specs=[
            pl.BlockSpec((RB, NC), lambda b: (b, 0)),
            pl.BlockSpec((RB, D), lambda b: (b, 0)),
            pl.BlockSpec((D, D), lambda b: (0, 0)),
            pl.BlockSpec((1, D), lambda b: (0, 0)),
        ],
        out_specs=[
            pl.BlockSpec((RB, 1), lambda b: (b, 0)),
            pl.BlockSpec((RB, D), lambda b: (b, 0)),
            pl.BlockSpec((RB, D), lambda b: (b, 0)),
        ],
        out_shape=[
            jax.ShapeDtypeStruct((N, 1), _f32),
            jax.ShapeDtypeStruct((N, D), _f32),
            jax.ShapeDtypeStruct((N, D), _f32),
        ],
    )(degp, x, w_in, b_in)


def _bn_from_acc(acc_ref):
    mean = acc_ref[0:1, :] / float(N)
    var = acc_ref[1:2, :] / float(N) - mean * mean
    return mean, lax.rsqrt(var + 1e-5)


def _layer_phase1(bm, part_ref, dis_ref, w_ref, b_ref, tbuf_ref, acc_ref):
    b = pl.program_id(0)
    a = (part_ref[0] + part_ref[1]) * dis_ref[...]
    t = lax.dot_general(a, w_ref[...], (((1,), (1,)), ((), ())),
                        preferred_element_type=_f32) + b_ref[...]
    tbuf_ref[bm] = t

    @pl.when(b == 0)
    def _():
        acc_ref[...] = jnp.zeros_like(acc_ref)

    acc_ref[0:1, :] += jnp.sum(t, axis=0, keepdims=True)
    acc_ref[1:2, :] += jnp.sum(t * t, axis=0, keepdims=True)


def _tc_layer_body(part_ref, dis_ref, w_ref, b_ref, h_ref, g_ref, be_ref,
                   hn_ref, y_ref, tbuf_ref, acc_ref):
    b = pl.program_id(0)
    bm = lax.rem(b, GRID)

    @pl.when(b < GRID)
    def _():
        _layer_phase1(bm, part_ref, dis_ref, w_ref, b_ref, tbuf_ref, acc_ref)

    @pl.when(b >= GRID)
    def _():
        mean, inv = _bn_from_acc(acc_ref)
        tn = (tbuf_ref[bm] - mean) * inv * g_ref[...] + be_ref[...]
        hn = jnp.maximum(tn, 0.0) + h_ref[...]
        hn_ref[...] = hn
        y_ref[...] = hn * dis_ref[...]


def _tc_layer(partial, dis, w, bvec, h, gamma, beta):
    return pl.pallas_call(
        _tc_layer_body,
        grid=(2 * GRID,),
        in_specs=[
            pl.BlockSpec((NC, RB, D), lambda b: (0, lax.min(b, GRID - 1), 0)),
            pl.BlockSpec((RB, 1), lambda b: (lax.rem(b, GRID), 0)),
            pl.BlockSpec((D, D), lambda b: (0, 0)),
            pl.BlockSpec((1, D), lambda b: (0, 0)),
            pl.BlockSpec((RB, D), lambda b: (lax.max(b - GRID, 0), 0)),
            pl.BlockSpec((1, D), lambda b: (0, 0)),
            pl.BlockSpec((1, D), lambda b: (0, 0)),
        ],
        out_specs=[
            pl.BlockSpec((RB, D), lambda b: (lax.rem(b, GRID), 0)),
            pl.BlockSpec((RB, D), lambda b: (lax.rem(b, GRID), 0)),
        ],
        out_shape=[
            jax.ShapeDtypeStruct((N, D), _f32),
            jax.ShapeDtypeStruct((N, D), _f32),
        ],
        scratch_shapes=[
            pltpu.VMEM((GRID, RB, D), _f32),
            pltpu.VMEM((2, D), _f32),
        ],
    )(partial, dis, w, bvec, h, gamma, beta)


def _tc_layer_out_body(part_ref, dis_ref, w_ref, b_ref, h_ref, g_ref, be_ref,
                       wo_ref, bo_ref, out_ref, tbuf_ref, acc_ref):
    b = pl.program_id(0)
    bm = lax.rem(b, GRID)

    @pl.when(b < GRID)
    def _():
        _layer_phase1(bm, part_ref, dis_ref, w_ref, b_ref, tbuf_ref, acc_ref)

    @pl.when(b >= GRID)
    def _():
        mean, inv = _bn_from_acc(acc_ref)
        tn = (tbuf_ref[bm] - mean) * inv * g_ref[...] + be_ref[...]
        hn = jnp.maximum(tn, 0.0) + h_ref[...]
        out_ref[...] = lax.dot_general(
            hn, wo_ref[...], (((1,), (1,)), ((), ())),
            preferred_element_type=_f32) + bo_ref[...]


def _tc_layer_out(partial, dis, w, bvec, h, gamma, beta, w_out, b_out):
    return pl.pallas_call(
        _tc_layer_out_body,
        grid=(2 * GRID,),
        in_specs=[
            pl.BlockSpec((NC, RB, D), lambda b: (0, lax.min(b, GRID - 1), 0)),
            pl.BlockSpec((RB, 1), lambda b: (lax.rem(b, GRID), 0)),
            pl.BlockSpec((D, D), lambda b: (0, 0)),
            pl.BlockSpec((1, D), lambda b: (0, 0)),
            pl.BlockSpec((RB, D), lambda b: (lax.max(b - GRID, 0), 0)),
            pl.BlockSpec((1, D), lambda b: (0, 0)),
            pl.BlockSpec((1, D), lambda b: (0, 0)),
            pl.BlockSpec((D, D), lambda b: (0, 0)),
            pl.BlockSpec((1, D), lambda b: (0, 0)),
        ],
        out_specs=pl.BlockSpec((RB, D), lambda b: (lax.rem(b, GRID), 0)),
        out_shape=jax.ShapeDtypeStruct((N, D), _f32),
        scratch_shapes=[
            pltpu.VMEM((GRID, RB, D), _f32),
            pltpu.VMEM((2, D), _f32),
        ],
    )(partial, dis, w, bvec, h, gamma, beta, w_out, b_out)


def kernel(x, edge_index, W_in, b_in, W1, b1, gamma1, beta1,
           W2, b2, gamma2, beta2, W_out, b_out):
    ei = edge_index.astype(jnp.int32)
    rows = ei[0].reshape(NW, NCHD, CHD)
    epacked = (ei[0] * (1 << 14) + ei[1]).reshape(NW, EPT)
    ones_e = jnp.ones((CHD,), _f32)
    zeros_n = jnp.zeros((ROWB,), _f32)
    zeros_zd = jnp.zeros((ROWB, D), _f32)

    degp = _sc_deg(rows, ones_e, zeros_n)
    dis, h, y = _tc_prep(degp.reshape(NC, N).T, x, W_in, b_in.reshape(1, D))

    part = _sc_spmm(epacked, y, zeros_zd)
    h, y = _tc_layer(part, dis, W1, b1.reshape(1, D), h,
                     gamma1.reshape(1, D), beta1.reshape(1, D))

    part = _sc_spmm(epacked, y, zeros_zd)
    out = _tc_layer_out(part, dis, W2, b2.reshape(1, D), h,
                        gamma2.reshape(1, D), beta2.reshape(1, D),
                        W_out, b_out.reshape(1, D))
    return out


# speedup vs baseline: 1.0928x; 1.0381x over previous
import functools
import jax
import jax.numpy as jnp
from jax import lax
from jax.experimental import pallas as pl
from jax.experimental.pallas import tpu as pltpu
from jax.experimental.pallas import tpu_sc as plsc

N = 10000
D = 128
E = 320000
NC = 2
NS = 16
NW = NC * NS
EPT = E // NW
CH = 80
NCHUNK = EPT // CH
CHD = 80
NCHD = EPT // CHD
ROWB = 1000
ZB = 40

_f32 = jnp.float32

_sc_mesh = plsc.VectorSubcoreMesh(core_axis_name="c", subcore_axis_name="s")


@functools.partial(
    pl.kernel,
    mesh=_sc_mesh,
    out_type=jax.ShapeDtypeStruct((NC * N,), _f32),
    scratch_types=[
        pltpu.VMEM((NCHD, CHD), jnp.int32),
        pltpu.VMEM((CHD,), _f32),
        pltpu.VMEM((ROWB,), _f32),
        pltpu.VMEM_SHARED((N,), _f32),
    ],
)
def _sc_deg(edges_hbm, ones_hbm, zeros_hbm, out_hbm, rowv, onesv, stg, degs):
    cid = lax.axis_index("c")
    sid = lax.axis_index("s")
    wid = cid * NS + sid

    pltpu.sync_copy(edges_hbm.at[wid], rowv)
    pltpu.sync_copy(ones_hbm, onesv)

    @pl.when(sid < N // ROWB)
    def _():
        pltpu.sync_copy(zeros_hbm, stg)
        pltpu.sync_copy(stg, degs.at[pl.ds(sid * ROWB, ROWB)])

    plsc.subcore_barrier()

    def body(g, carry):
        pltpu.sync_copy(onesv, degs.at[rowv.at[g]], add=True)
        return carry

    lax.fori_loop(0, NCHD, body, 0, unroll=False)

    plsc.subcore_barrier()

    @pl.when(sid < N // ROWB)
    def _():
        pltpu.sync_copy(degs.at[pl.ds(sid * ROWB, ROWB)], stg)
        pltpu.sync_copy(stg, out_hbm.at[pl.ds(cid * N + sid * ROWB, ROWB)])


@functools.partial(
    pl.kernel,
    mesh=_sc_mesh,
    out_type=jax.ShapeDtypeStruct((NC, N, D), _f32),
    scratch_types=[
        pltpu.VMEM((EPT,), jnp.int32),
        pltpu.VMEM((CH,), jnp.int32),
        pltpu.VMEM((CH,), jnp.int32),
        pltpu.VMEM((CH,), jnp.int32),
        pltpu.VMEM((CH,), jnp.int32),
        pltpu.VMEM((CH, D), _f32),
        pltpu.VMEM((CH, D), _f32),
        pltpu.VMEM_SHARED((N, D), _f32),
        pltpu.SemaphoreType.DMA,
        pltpu.SemaphoreType.DMA,
    ],
)
def _sc_spmm(epk_hbm, y_hbm, zeros_hbm, out_hbm,
             pk, rowca, colca, rowcb, colcb, bufa, bufb, agg, sema, semb):
    cid = lax.axis_index("c")
    sid = lax.axis_index("s")
    wid = cid * NS + sid

    pltpu.sync_copy(epk_hbm.at[wid], pk)

    @pl.when(sid < N // ROWB)
    def _():
        pltpu.sync_copy(zeros_hbm, agg.at[pl.ds(sid * ROWB, ROWB)])

    plsc.subcore_barrier()

    def unpack(g, rowc, colc):
        for k in range(CH // 16):
            v = pk[pl.ds(g * CH + 16 * k, 16)]
            rowc[pl.ds(16 * k, 16)] = lax.shift_right_logical(v, 14)
            colc[pl.ds(16 * k, 16)] = lax.bitwise_and(v, (1 << 14) - 1)

    unpack(0, rowca, colca)
    pltpu.make_async_copy(y_hbm.at[colca], bufa, sema).start()

    def body(t, carry):
        ga = 2 * t
        unpack(ga + 1, rowcb, colcb)
        pltpu.make_async_copy(y_hbm.at[colcb], bufb, semb).start()
        pltpu.make_async_copy(y_hbm.at[colca], bufa, sema).wait()
        pltpu.sync_copy(bufa, agg.at[rowca], add=True)
        unpack(ga + 2, rowca, colca)
        pltpu.make_async_copy(y_hbm.at[colca], bufa, sema).start()
        pltpu.make_async_copy(y_hbm.at[colcb], bufb, semb).wait()
        pltpu.sync_copy(bufb, agg.at[rowcb], add=True)
        return carry

    lax.fori_loop(0, (NCHUNK - 1) // 2, body, 0, unroll=False)

    pltpu.make_async_copy(y_hbm.at[colca], bufa, sema).wait()
    pltpu.sync_copy(bufa, agg.at[rowca], add=True)

    plsc.subcore_barrier()

    @pl.when(sid < N // ROWB)
    def _():
        pltpu.sync_copy(agg.at[pl.ds(sid * ROWB, ROWB)],
                        out_hbm.at[cid, pl.ds(sid * ROWB, ROWB)])


RB = 2000
GRID = N // RB


def _tc_prep_body(degp_ref, x_ref, w_ref, b_ref, dis_ref, h_ref, y_ref):
    deg = degp_ref[:, 0:1] + degp_ref[:, 1:2]
    dis = jnp.where(deg > 0.0,
                    lax.rsqrt(jnp.maximum(deg, 1e-12)), 0.0)
    h = lax.dot_general(x_ref[...], w_ref[...],
                        (((1,), (1,)), ((), ())),
                        preferred_element_type=_f32) + b_ref[...]
    dis_ref[...] = dis
    h_ref[...] = h
    y_ref[...] = h * dis


def _tc_prep(degp, x, w_in, b_in):
    return pl.pallas_call(
        _tc_prep_body,
        grid=(GRID,),
        in_specs=[
            pl.BlockSpec((RB, NC), lambda b: (b, 0)),
            pl.BlockSpec((RB, D), lambda b: (b, 0)),
            pl.BlockSpec((D, D), lambda b: (0, 0)),
            pl.BlockSpec((1, D), lambda b: (0, 0)),
        ],
        out_specs=[
            pl.BlockSpec((RB, 1), lambda b: (b, 0)),
            pl.BlockSpec((RB, D), lambda b: (b, 0)),
            pl.BlockSpec((RB, D), lambda b: (b, 0)),
        ],
        out_shape=[
            jax.ShapeDtypeStruct((N, 1), _f32),
            jax.ShapeDtypeStruct((N, D), _f32),
            jax.ShapeDtypeStruct((N, D), _f32),
        ],
    )(degp, x, w_in, b_in)


def _bn_from_acc(acc_ref):
    mean = acc_ref[0:1, :] / float(N)
    var = acc_ref[1:2, :] / float(N) - mean * mean
    return mean, lax.rsqrt(var + 1e-5)


def _layer_phase1(bm, part_ref, dis_ref, w_ref, b_ref, tbuf_ref, acc_ref):
    b = pl.program_id(0)
    a = (part_ref[0] + part_ref[1]) * dis_ref[...]
    t = lax.dot_general(a, w_ref[...], (((1,), (1,)), ((), ())),
                        preferred_element_type=_f32) + b_ref[...]
    tbuf_ref[bm] = t

    @pl.when(b == 0)
    def _():
        acc_ref[...] = jnp.zeros_like(acc_ref)

    acc_ref[0:1, :] += jnp.sum(t, axis=0, keepdims=True)
    acc_ref[1:2, :] += jnp.sum(t * t, axis=0, keepdims=True)


def _tc_layer_body(part_ref, dis_ref, w_ref, b_ref, h_ref, g_ref, be_ref,
                   hn_ref, y_ref, tbuf_ref, acc_ref):
    b = pl.program_id(0)
    bm = lax.rem(b, GRID)

    @pl.when(b < GRID)
    def _():
        _layer_phase1(bm, part_ref, dis_ref, w_ref, b_ref, tbuf_ref, acc_ref)

    @pl.when(b >= GRID)
    def _():
        mean, inv = _bn_from_acc(acc_ref)
        tn = (tbuf_ref[bm] - mean) * inv * g_ref[...] + be_ref[...]
        hn = jnp.maximum(tn, 0.0) + h_ref[...]
        hn_ref[...] = hn
        y_ref[...] = hn * dis_ref[...]


def _tc_layer(partial, dis, w, bvec, h, gamma, beta):
    return pl.pallas_call(
        _tc_layer_body,
        grid=(2 * GRID,),
        in_specs=[
            pl.BlockSpec((NC, RB, D), lambda b: (0, lax.min(b, GRID - 1), 0)),
            pl.BlockSpec((RB, 1), lambda b: (lax.rem(b, GRID), 0)),
            pl.BlockSpec((D, D), lambda b: (0, 0)),
            pl.BlockSpec((1, D), lambda b: (0, 0)),
            pl.BlockSpec((RB, D), lambda b: (lax.max(b - GRID, 0), 0)),
            pl.BlockSpec((1, D), lambda b: (0, 0)),
            pl.BlockSpec((1, D), lambda b: (0, 0)),
        ],
        out_specs=[
            pl.BlockSpec((RB, D), lambda b: (lax.rem(b, GRID), 0)),
            pl.BlockSpec((RB, D), lambda b: (lax.rem(b, GRID), 0)),
        ],
        out_shape=[
            jax.ShapeDtypeStruct((N, D), _f32),
            jax.ShapeDtypeStruct((N, D), _f32),
        ],
        scratch_shapes=[
            pltpu.VMEM((GRID, RB, D), _f32),
            pltpu.VMEM((2, D), _f32),
        ],
    )(partial, dis, w, bvec, h, gamma, beta)


def _tc_layer_out_body(part_ref, dis_ref, w_ref, b_ref, h_ref, g_ref, be_ref,
                       wo_ref, bo_ref, out_ref, tbuf_ref, acc_ref):
    b = pl.program_id(0)
    bm = lax.rem(b, GRID)

    @pl.when(b < GRID)
    def _():
        _layer_phase1(bm, part_ref, dis_ref, w_ref, b_ref, tbuf_ref, acc_ref)

    @pl.when(b >= GRID)
    def _():
        mean, inv = _bn_from_acc(acc_ref)
        tn = (tbuf_ref[bm] - mean) * inv * g_ref[...] + be_ref[...]
        hn = jnp.maximum(tn, 0.0) + h_ref[...]
        out_ref[...] = lax.dot_general(
            hn, wo_ref[...], (((1,), (1,)), ((), ())),
            preferred_element_type=_f32) + bo_ref[...]


def _tc_layer_out(partial, dis, w, bvec, h, gamma, beta, w_out, b_out):
    return pl.pallas_call(
        _tc_layer_out_body,
        grid=(2 * GRID,),
        in_specs=[
            pl.BlockSpec((NC, RB, D), lambda b: (0, lax.min(b, GRID - 1), 0)),
            pl.BlockSpec((RB, 1), lambda b: (lax.rem(b, GRID), 0)),
            pl.BlockSpec((D, D), lambda b: (0, 0)),
            pl.BlockSpec((1, D), lambda b: (0, 0)),
            pl.BlockSpec((RB, D), lambda b: (lax.max(b - GRID, 0), 0)),
            pl.BlockSpec((1, D), lambda b: (0, 0)),
            pl.BlockSpec((1, D), lambda b: (0, 0)),
            pl.BlockSpec((D, D), lambda b: (0, 0)),
            pl.BlockSpec((1, D), lambda b: (0, 0)),
        ],
        out_specs=pl.BlockSpec((RB, D), lambda b: (lax.rem(b, GRID), 0)),
        out_shape=jax.ShapeDtypeStruct((N, D), _f32),
        scratch_shapes=[
            pltpu.VMEM((GRID, RB, D), _f32),
            pltpu.VMEM((2, D), _f32),
        ],
    )(partial, dis, w, bvec, h, gamma, beta, w_out, b_out)


def kernel(x, edge_index, W_in, b_in, W1, b1, gamma1, beta1,
           W2, b2, gamma2, beta2, W_out, b_out):
    ei = edge_index.astype(jnp.int32)
    rows = ei[0].reshape(NW, NCHD, CHD)
    epacked = (ei[0] * (1 << 14) + ei[1]).reshape(NW, EPT)
    ones_e = jnp.ones((CHD,), _f32)
    zeros_n = jnp.zeros((ROWB,), _f32)
    zeros_zd = jnp.zeros((ROWB, D), _f32)

    degp = _sc_deg(rows, ones_e, zeros_n)
    dis, h, y = _tc_prep(degp.reshape(NC, N).T, x, W_in, b_in.reshape(1, D))

    part = _sc_spmm(epacked, y, zeros_zd)
    h, y = _tc_layer(part, dis, W1, b1.reshape(1, D), h,
                     gamma1.reshape(1, D), beta1.reshape(1, D))

    part = _sc_spmm(epacked, y, zeros_zd)
    out = _tc_layer_out(part, dis, W2, b2.reshape(1, D), h,
                        gamma2.reshape(1, D), beta2.reshape(1, D),
                        W_out, b_out.reshape(1, D))
    return out


# speedup vs baseline: 1.1159x; 1.0211x over previous
import functools
import jax
import jax.numpy as jnp
from jax import lax
from jax.experimental import pallas as pl
from jax.experimental.pallas import tpu as pltpu
from jax.experimental.pallas import tpu_sc as plsc

N = 10000
D = 128
E = 320000
NC = 2
NS = 16
NW = NC * NS
EPT = E // NW
CH = 80
NCHUNK = EPT // CH
CHD = 80
NCHD = EPT // CHD
ROWB = 1000
ZB = 40

_f32 = jnp.float32

_sc_mesh = plsc.VectorSubcoreMesh(core_axis_name="c", subcore_axis_name="s")


@functools.partial(
    pl.kernel,
    mesh=_sc_mesh,
    out_type=jax.ShapeDtypeStruct((NC * N,), _f32),
    scratch_types=[
        pltpu.VMEM((NCHD, CHD), jnp.int32),
        pltpu.VMEM((CHD,), _f32),
        pltpu.VMEM((ROWB,), _f32),
        pltpu.VMEM_SHARED((N,), _f32),
    ],
)
def _sc_deg(edges_hbm, ones_hbm, zeros_hbm, out_hbm, rowv, onesv, stg, degs):
    cid = lax.axis_index("c")
    sid = lax.axis_index("s")
    wid = cid * NS + sid

    pltpu.sync_copy(edges_hbm.at[wid], rowv)
    pltpu.sync_copy(ones_hbm, onesv)

    @pl.when(sid < N // ROWB)
    def _():
        pltpu.sync_copy(zeros_hbm, stg)
        pltpu.sync_copy(stg, degs.at[pl.ds(sid * ROWB, ROWB)])

    plsc.subcore_barrier()

    def body(g, carry):
        pltpu.sync_copy(onesv, degs.at[rowv.at[g]], add=True)
        return carry

    lax.fori_loop(0, NCHD, body, 0, unroll=False)

    plsc.subcore_barrier()

    @pl.when(sid < N // ROWB)
    def _():
        pltpu.sync_copy(degs.at[pl.ds(sid * ROWB, ROWB)], stg)
        pltpu.sync_copy(stg, out_hbm.at[pl.ds(cid * N + sid * ROWB, ROWB)])


@functools.partial(
    pl.kernel,
    mesh=_sc_mesh,
    out_type=jax.ShapeDtypeStruct((NC, N, D), _f32),
    scratch_types=[
        pltpu.VMEM((EPT,), jnp.int32),
        pltpu.VMEM((CH,), jnp.int32),
        pltpu.VMEM((CH,), jnp.int32),
        pltpu.VMEM((CH,), jnp.int32),
        pltpu.VMEM((CH,), jnp.int32),
        pltpu.VMEM((CH, D), _f32),
        pltpu.VMEM((CH, D), _f32),
        pltpu.VMEM_SHARED((N, D), _f32),
        pltpu.SemaphoreType.DMA,
        pltpu.SemaphoreType.DMA,
    ],
)
def _sc_spmm(epk_hbm, y_hbm, zeros_hbm, out_hbm,
             pk, rowca, colca, rowcb, colcb, bufa, bufb, agg, sema, semb):
    cid = lax.axis_index("c")
    sid = lax.axis_index("s")
    wid = cid * NS + sid

    pltpu.sync_copy(epk_hbm.at[wid], pk)

    @pl.when(sid < N // ROWB)
    def _():
        pltpu.sync_copy(zeros_hbm, agg.at[pl.ds(sid * ROWB, ROWB)])

    plsc.subcore_barrier()

    def unpack(g, rowc, colc):
        for k in range(CH // 16):
            v = pk[pl.ds(g * CH + 16 * k, 16)]
            rowc[pl.ds(16 * k, 16)] = lax.shift_right_logical(v, 14)
            colc[pl.ds(16 * k, 16)] = lax.bitwise_and(v, (1 << 14) - 1)

    unpack(0, rowca, colca)
    pltpu.make_async_copy(y_hbm.at[colca], bufa, sema).start()

    def body(t, carry):
        ga = 2 * t
        unpack(ga + 1, rowcb, colcb)
        pltpu.make_async_copy(y_hbm.at[colcb], bufb, semb).start()
        pltpu.make_async_copy(y_hbm.at[colca], bufa, sema).wait()
        pltpu.sync_copy(bufa, agg.at[rowca], add=True)
        unpack(ga + 2, rowca, colca)
        pltpu.make_async_copy(y_hbm.at[colca], bufa, sema).start()
        pltpu.make_async_copy(y_hbm.at[colcb], bufb, semb).wait()
        pltpu.sync_copy(bufb, agg.at[rowcb], add=True)
        return carry

    lax.fori_loop(0, (NCHUNK - 1) // 2, body, 0, unroll=False)

    pltpu.make_async_copy(y_hbm.at[colca], bufa, sema).wait()
    pltpu.sync_copy(bufa, agg.at[rowca], add=True)

    plsc.subcore_barrier()

    @pl.when(sid < N // ROWB)
    def _():
        pltpu.sync_copy(agg.at[pl.ds(sid * ROWB, ROWB)],
                        out_hbm.at[cid, pl.ds(sid * ROWB, ROWB)])


RB = 5000
GRID = N // RB


def _tc_prep_body(degp_ref, x_ref, w_ref, b_ref, dis_ref, h_ref, y_ref):
    deg = degp_ref[:, 0:1] + degp_ref[:, 1:2]
    dis = jnp.where(deg > 0.0,
                    lax.rsqrt(jnp.maximum(deg, 1e-12)), 0.0)
    h = lax.dot_general(x_ref[...], w_ref[...],
                        (((1,), (1,)), ((), ())),
                        preferred_element_type=_f32) + b_ref[...]
    dis_ref[...] = dis
    h_ref[...] = h
    y_ref[...] = h * dis


def _tc_prep(degp, x, w_in, b_in):
    return pl.pallas_call(
        _tc_prep_body,
        grid=(GRID,),
        in_specs=[
            pl.BlockSpec((RB, NC), lambda b: (b, 0)),
            pl.BlockSpec((RB, D), lambda b: (b, 0)),
            pl.BlockSpec((D, D), lambda b: (0, 0)),
            pl.BlockSpec((1, D), lambda b: (0, 0)),
        ],
        out_specs=[
            pl.BlockSpec((RB, 1), lambda b: (b, 0)),
            pl.BlockSpec((RB, D), lambda b: (b, 0)),
            pl.BlockSpec((RB, D), lambda b: (b, 0)),
        ],
        out_shape=[
            jax.ShapeDtypeStruct((N, 1), _f32),
            jax.ShapeDtypeStruct((N, D), _f32),
            jax.ShapeDtypeStruct((N, D), _f32),
        ],
    )(degp, x, w_in, b_in)


def _bn_from_acc(acc_ref):
    mean = acc_ref[0:1, :] / float(N)
    var = acc_ref[1:2, :] / float(N) - mean * mean
    return mean, lax.rsqrt(var + 1e-5)


def _layer_phase1(bm, part_ref, dis_ref, w_ref, b_ref, tbuf_ref, acc_ref):
    b = pl.program_id(0)
    a = (part_ref[0] + part_ref[1]) * dis_ref[...]
    t = lax.dot_general(a, w_ref[...], (((1,), (1,)), ((), ())),
                        preferred_element_type=_f32) + b_ref[...]
    tbuf_ref[bm] = t

    @pl.when(b == 0)
    def _():
        acc_ref[...] = jnp.zeros_like(acc_ref)

    acc_ref[0:1, :] += jnp.sum(t, axis=0, keepdims=True)
    acc_ref[1:2, :] += jnp.sum(t * t, axis=0, keepdims=True)


def _tc_layer_body(part_ref, dis_ref, w_ref, b_ref, h_ref, g_ref, be_ref,
                   hn_ref, y_ref, tbuf_ref, acc_ref):
    b = pl.program_id(0)
    bm = lax.rem(b, GRID)

    @pl.when(b < GRID)
    def _():
        _layer_phase1(bm, part_ref, dis_ref, w_ref, b_ref, tbuf_ref, acc_ref)

    @pl.when(b >= GRID)
    def _():
        mean, inv = _bn_from_acc(acc_ref)
        tn = (tbuf_ref[bm] - mean) * inv * g_ref[...] + be_ref[...]
        hn = jnp.maximum(tn, 0.0) + h_ref[...]
        hn_ref[...] = hn
        y_ref[...] = hn * dis_ref[...]


def _tc_layer(partial, dis, w, bvec, h, gamma, beta):
    return pl.pallas_call(
        _tc_layer_body,
        grid=(2 * GRID,),
        in_specs=[
            pl.BlockSpec((NC, RB, D), lambda b: (0, lax.min(b, GRID - 1), 0)),
            pl.BlockSpec((RB, 1), lambda b: (lax.rem(b, GRID), 0)),
            pl.BlockSpec((D, D), lambda b: (0, 0)),
            pl.BlockSpec((1, D), lambda b: (0, 0)),
            pl.BlockSpec((RB, D), lambda b: (lax.max(b - GRID, 0), 0)),
            pl.BlockSpec((1, D), lambda b: (0, 0)),
            pl.BlockSpec((1, D), lambda b: (0, 0)),
        ],
        out_specs=[
            pl.BlockSpec((RB, D), lambda b: (lax.rem(b, GRID), 0)),
            pl.BlockSpec((RB, D), lambda b: (lax.rem(b, GRID), 0)),
        ],
        out_shape=[
            jax.ShapeDtypeStruct((N, D), _f32),
            jax.ShapeDtypeStruct((N, D), _f32),
        ],
        scratch_shapes=[
            pltpu.VMEM((GRID, RB, D), _f32),
            pltpu.VMEM((2, D), _f32),
        ],
    )(partial, dis, w, bvec, h, gamma, beta)


def _tc_layer_out_body(part_ref, dis_ref, w_ref, b_ref, h_ref, g_ref, be_ref,
                       wo_ref, bo_ref, out_ref, tbuf_ref, acc_ref):
    b = pl.program_id(0)
    bm = lax.rem(b, GRID)

    @pl.when(b < GRID)
    def _():
        _layer_phase1(bm, part_ref, dis_ref, w_ref, b_ref, tbuf_ref, acc_ref)

    @pl.when(b >= GRID)
    def _():
        mean, inv = _bn_from_acc(acc_ref)
        tn = (tbuf_ref[bm] - mean) * inv * g_ref[...] + be_ref[...]
        hn = jnp.maximum(tn, 0.0) + h_ref[...]
        out_ref[...] = lax.dot_general(
            hn, wo_ref[...], (((1,), (1,)), ((), ())),
            preferred_element_type=_f32) + bo_ref[...]


def _tc_layer_out(partial, dis, w, bvec, h, gamma, beta, w_out, b_out):
    return pl.pallas_call(
        _tc_layer_out_body,
        grid=(2 * GRID,),
        in_specs=[
            pl.BlockSpec((NC, RB, D), lambda b: (0, lax.min(b, GRID - 1), 0)),
            pl.BlockSpec((RB, 1), lambda b: (lax.rem(b, GRID), 0)),
            pl.BlockSpec((D, D), lambda b: (0, 0)),
            pl.BlockSpec((1, D), lambda b: (0, 0)),
            pl.BlockSpec((RB, D), lambda b: (lax.max(b - GRID, 0), 0)),
            pl.BlockSpec((1, D), lambda b: (0, 0)),
            pl.BlockSpec((1, D), lambda b: (0, 0)),
            pl.BlockSpec((D, D), lambda b: (0, 0)),
            pl.BlockSpec((1, D), lambda b: (0, 0)),
        ],
        out_specs=pl.BlockSpec((RB, D), lambda b: (lax.rem(b, GRID), 0)),
        out_shape=jax.ShapeDtypeStruct((N, D), _f32),
        scratch_shapes=[
            pltpu.VMEM((GRID, RB, D), _f32),
            pltpu.VMEM((2, D), _f32),
        ],
    )(partial, dis, w, bvec, h, gamma, beta, w_out, b_out)


def kernel(x, edge_index, W_in, b_in, W1, b1, gamma1, beta1,
           W2, b2, gamma2, beta2, W_out, b_out):
    ei = edge_index.astype(jnp.int32)
    rows = ei[0].reshape(NW, NCHD, CHD)
    epacked = (ei[0] * (1 << 14) + ei[1]).reshape(NW, EPT)
    ones_e = jnp.ones((CHD,), _f32)
    zeros_n = jnp.zeros((ROWB,), _f32)
    zeros_zd = jnp.zeros((ROWB, D), _f32)

    degp = _sc_deg(rows, ones_e, zeros_n)
    dis, h, y = _tc_prep(degp.reshape(NC, N).T, x, W_in, b_in.reshape(1, D))

    part = _sc_spmm(epacked, y, zeros_zd)
    h, y = _tc_layer(part, dis, W1, b1.reshape(1, D), h,
                     gamma1.reshape(1, D), beta1.reshape(1, D))

    part = _sc_spmm(epacked, y, zeros_zd)
    out = _tc_layer_out(part, dis, W2, b2.reshape(1, D), h,
                        gamma2.reshape(1, D), beta2.reshape(1, D),
                        W_out, b_out.reshape(1, D))
    return out


# speedup vs baseline: 1.1180x; 1.0019x over previous
import functools
import jax
import jax.numpy as jnp
from jax import lax
from jax.experimental import pallas as pl
from jax.experimental.pallas import tpu as pltpu
from jax.experimental.pallas import tpu_sc as plsc

N = 10000
D = 128
E = 320000
NC = 2
NS = 16
NW = NC * NS
EPT = E // NW
CH = 80
NCHUNK = EPT // CH
CHD = 80
NCHD = EPT // CHD
ROWB = 1000
ZB = 40

_f32 = jnp.float32

_sc_mesh = plsc.VectorSubcoreMesh(core_axis_name="c", subcore_axis_name="s")


@functools.partial(
    pl.kernel,
    mesh=_sc_mesh,
    out_type=jax.ShapeDtypeStruct((NC * N,), _f32),
    scratch_types=[
        pltpu.VMEM((NCHD, CHD), jnp.int32),
        pltpu.VMEM((CHD,), _f32),
        pltpu.VMEM((ROWB,), _f32),
        pltpu.VMEM_SHARED((N,), _f32),
    ],
)
def _sc_deg(edges_hbm, ones_hbm, zeros_hbm, out_hbm, rowv, onesv, stg, degs):
    cid = lax.axis_index("c")
    sid = lax.axis_index("s")
    wid = cid * NS + sid

    pltpu.sync_copy(edges_hbm.at[wid], rowv)
    pltpu.sync_copy(ones_hbm, onesv)

    @pl.when(sid < N // ROWB)
    def _():
        pltpu.sync_copy(zeros_hbm, stg)
        pltpu.sync_copy(stg, degs.at[pl.ds(sid * ROWB, ROWB)])

    plsc.subcore_barrier()

    def body(g, carry):
        pltpu.sync_copy(onesv, degs.at[rowv.at[g]], add=True)
        return carry

    lax.fori_loop(0, NCHD, body, 0, unroll=False)

    plsc.subcore_barrier()

    @pl.when(sid < N // ROWB)
    def _():
        pltpu.sync_copy(degs.at[pl.ds(sid * ROWB, ROWB)], stg)
        pltpu.sync_copy(stg, out_hbm.at[pl.ds(cid * N + sid * ROWB, ROWB)])


@functools.partial(
    pl.kernel,
    mesh=_sc_mesh,
    out_type=jax.ShapeDtypeStruct((NC, N, D), _f32),
    scratch_types=[
        pltpu.VMEM((EPT,), jnp.int32),
        pltpu.VMEM((CH,), jnp.int32),
        pltpu.VMEM((CH,), jnp.int32),
        pltpu.VMEM((CH,), jnp.int32),
        pltpu.VMEM((CH,), jnp.int32),
        pltpu.VMEM((CH, D), _f32),
        pltpu.VMEM((CH, D), _f32),
        pltpu.VMEM_SHARED((N, D), _f32),
        pltpu.SemaphoreType.DMA,
        pltpu.SemaphoreType.DMA,
    ],
)
def _sc_spmm(epk_hbm, y_hbm, zeros_hbm, out_hbm,
             pk, rowca, colca, rowcb, colcb, bufa, bufb, agg, sema, semb):
    cid = lax.axis_index("c")
    sid = lax.axis_index("s")
    wid = cid * NS + sid

    pltpu.sync_copy(epk_hbm.at[wid], pk)

    @pl.when(sid < N // ROWB)
    def _():
        pltpu.sync_copy(zeros_hbm, agg.at[pl.ds(sid * ROWB, ROWB)])

    plsc.subcore_barrier()

    def unpack(g, rowc, colc):
        for k in range(CH // 16):
            v = pk[pl.ds(g * CH + 16 * k, 16)]
            rowc[pl.ds(16 * k, 16)] = lax.shift_right_logical(v, 14)
            colc[pl.ds(16 * k, 16)] = lax.bitwise_and(v, (1 << 14) - 1)

    unpack(0, rowca, colca)
    pltpu.make_async_copy(y_hbm.at[colca], bufa, sema).start()

    def body(t, carry):
        ga = 2 * t
        unpack(ga + 1, rowcb, colcb)
        pltpu.make_async_copy(y_hbm.at[colcb], bufb, semb).start()
        pltpu.make_async_copy(y_hbm.at[colca], bufa, sema).wait()
        pltpu.sync_copy(bufa, agg.at[rowca], add=True)
        unpack(ga + 2, rowca, colca)
        pltpu.make_async_copy(y_hbm.at[colca], bufa, sema).start()
        pltpu.make_async_copy(y_hbm.at[colcb], bufb, semb).wait()
        pltpu.sync_copy(bufb, agg.at[rowcb], add=True)
        return carry

    lax.fori_loop(0, (NCHUNK - 1) // 2, body, 0, unroll=False)

    pltpu.make_async_copy(y_hbm.at[colca], bufa, sema).wait()
    pltpu.sync_copy(bufa, agg.at[rowca], add=True)

    plsc.subcore_barrier()

    @pl.when(sid < N // ROWB)
    def _():
        pltpu.sync_copy(agg.at[pl.ds(sid * ROWB, ROWB)],
                        out_hbm.at[cid, pl.ds(sid * ROWB, ROWB)])


RB = 10000
GRID = N // RB


def _tc_prep_body(degp_ref, x_ref, w_ref, b_ref, dis_ref, h_ref, y_ref):
    deg = degp_ref[:, 0:1] + degp_ref[:, 1:2]
    dis = jnp.where(deg > 0.0,
                    lax.rsqrt(jnp.maximum(deg, 1e-12)), 0.0)
    h = lax.dot_general(x_ref[...], w_ref[...],
                        (((1,), (1,)), ((), ())),
                        preferred_element_type=_f32) + b_ref[...]
    dis_ref[...] = dis
    h_ref[...] = h
    y_ref[...] = h * dis


def _tc_prep(degp, x, w_in, b_in):
    return pl.pallas_call(
        _tc_prep_body,
        grid=(GRID,),
        in_specs=[
            pl.BlockSpec((RB, NC), lambda b: (b, 0)),
            pl.BlockSpec((RB, D), lambda b: (b, 0)),
            pl.BlockSpec((D, D), lambda b: (0, 0)),
            pl.BlockSpec((1, D), lambda b: (0, 0)),
        ],
        out_specs=[
            pl.BlockSpec((RB, 1), lambda b: (b, 0)),
            pl.BlockSpec((RB, D), lambda b: (b, 0)),
            pl.BlockSpec((RB, D), lambda b: (b, 0)),
        ],
        out_shape=[
            jax.ShapeDtypeStruct((N, 1), _f32),
            jax.ShapeDtypeStruct((N, D), _f32),
            jax.ShapeDtypeStruct((N, D), _f32),
        ],
    )(degp, x, w_in, b_in)


def _bn_from_acc(acc_ref):
    mean = acc_ref[0:1, :] / float(N)
    var = acc_ref[1:2, :] / float(N) - mean * mean
    return mean, lax.rsqrt(var + 1e-5)


def _layer_phase1(bm, part_ref, dis_ref, w_ref, b_ref, tbuf_ref, acc_ref):
    b = pl.program_id(0)
    a = (part_ref[0] + part_ref[1]) * dis_ref[...]
    t = lax.dot_general(a, w_ref[...], (((1,), (1,)), ((), ())),
                        preferred_element_type=_f32) + b_ref[...]
    tbuf_ref[bm] = t

    @pl.when(b == 0)
    def _():
        acc_ref[...] = jnp.zeros_like(acc_ref)

    acc_ref[0:1, :] += jnp.sum(t, axis=0, keepdims=True)
    acc_ref[1:2, :] += jnp.sum(t * t, axis=0, keepdims=True)


def _tc_layer_body(part_ref, dis_ref, w_ref, b_ref, h_ref, g_ref, be_ref,
                   hn_ref, y_ref, tbuf_ref, acc_ref):
    b = pl.program_id(0)
    bm = lax.rem(b, GRID)

    @pl.when(b < GRID)
    def _():
        _layer_phase1(bm, part_ref, dis_ref, w_ref, b_ref, tbuf_ref, acc_ref)

    @pl.when(b >= GRID)
    def _():
        mean, inv = _bn_from_acc(acc_ref)
        tn = (tbuf_ref[bm] - mean) * inv * g_ref[...] + be_ref[...]
        hn = jnp.maximum(tn, 0.0) + h_ref[...]
        hn_ref[...] = hn
        y_ref[...] = hn * dis_ref[...]


def _tc_layer(partial, dis, w, bvec, h, gamma, beta):
    return pl.pallas_call(
        _tc_layer_body,
        grid=(2 * GRID,),
        in_specs=[
            pl.BlockSpec((NC, RB, D), lambda b: (0, lax.min(b, GRID - 1), 0)),
            pl.BlockSpec((RB, 1), lambda b: (lax.rem(b, GRID), 0)),
            pl.BlockSpec((D, D), lambda b: (0, 0)),
            pl.BlockSpec((1, D), lambda b: (0, 0)),
            pl.BlockSpec((RB, D), lambda b: (lax.max(b - GRID, 0), 0)),
            pl.BlockSpec((1, D), lambda b: (0, 0)),
            pl.BlockSpec((1, D), lambda b: (0, 0)),
        ],
        out_specs=[
            pl.BlockSpec((RB, D), lambda b: (lax.rem(b, GRID), 0)),
            pl.BlockSpec((RB, D), lambda b: (lax.rem(b, GRID), 0)),
        ],
        out_shape=[
            jax.ShapeDtypeStruct((N, D), _f32),
            jax.ShapeDtypeStruct((N, D), _f32),
        ],
        scratch_shapes=[
            pltpu.VMEM((GRID, RB, D), _f32),
            pltpu.VMEM((2, D), _f32),
        ],
    )(partial, dis, w, bvec, h, gamma, beta)


def _tc_layer_out_body(part_ref, dis_ref, w_ref, b_ref, h_ref, g_ref, be_ref,
                       wo_ref, bo_ref, out_ref, tbuf_ref, acc_ref):
    b = pl.program_id(0)
    bm = lax.rem(b, GRID)

    @pl.when(b < GRID)
    def _():
        _layer_phase1(bm, part_ref, dis_ref, w_ref, b_ref, tbuf_ref, acc_ref)

    @pl.when(b >= GRID)
    def _():
        mean, inv = _bn_from_acc(acc_ref)
        tn = (tbuf_ref[bm] - mean) * inv * g_ref[...] + be_ref[...]
        hn = jnp.maximum(tn, 0.0) + h_ref[...]
        out_ref[...] = lax.dot_general(
            hn, wo_ref[...], (((1,), (1,)), ((), ())),
            preferred_element_type=_f32) + bo_ref[...]


def _tc_layer_out(partial, dis, w, bvec, h, gamma, beta, w_out, b_out):
    return pl.pallas_call(
        _tc_layer_out_body,
        grid=(2 * GRID,),
        in_specs=[
            pl.BlockSpec((NC, RB, D), lambda b: (0, lax.min(b, GRID - 1), 0)),
            pl.BlockSpec((RB, 1), lambda b: (lax.rem(b, GRID), 0)),
            pl.BlockSpec((D, D), lambda b: (0, 0)),
            pl.BlockSpec((1, D), lambda b: (0, 0)),
            pl.BlockSpec((RB, D), lambda b: (lax.max(b - GRID, 0), 0)),
            pl.BlockSpec((1, D), lambda b: (0, 0)),
            pl.BlockSpec((1, D), lambda b: (0, 0)),
            pl.BlockSpec((D, D), lambda b: (0, 0)),
            pl.BlockSpec((1, D), lambda b: (0, 0)),
        ],
        out_specs=pl.BlockSpec((RB, D), lambda b: (lax.rem(b, GRID), 0)),
        out_shape=jax.ShapeDtypeStruct((N, D), _f32),
        scratch_shapes=[
            pltpu.VMEM((GRID, RB, D), _f32),
            pltpu.VMEM((2, D), _f32),
        ],
    )(partial, dis, w, bvec, h, gamma, beta, w_out, b_out)


def kernel(x, edge_index, W_in, b_in, W1, b1, gamma1, beta1,
           W2, b2, gamma2, beta2, W_out, b_out):
    ei = edge_index.astype(jnp.int32)
    rows = ei[0].reshape(NW, NCHD, CHD)
    epacked = (ei[0] * (1 << 14) + ei[1]).reshape(NW, EPT)
    ones_e = jnp.ones((CHD,), _f32)
    zeros_n = jnp.zeros((ROWB,), _f32)
    zeros_zd = jnp.zeros((ROWB, D), _f32)

    degp = _sc_deg(rows, ones_e, zeros_n)
    dis, h, y = _tc_prep(degp.reshape(NC, N).T, x, W_in, b_in.reshape(1, D))

    part = _sc_spmm(epacked, y, zeros_zd)
    h, y = _tc_layer(part, dis, W1, b1.reshape(1, D), h,
                     gamma1.reshape(1, D), beta1.reshape(1, D))

    part = _sc_spmm(epacked, y, zeros_zd)
    out = _tc_layer_out(part, dis, W2, b2.reshape(1, D), h,
                        gamma2.reshape(1, D), beta2.reshape(1, D),
                        W_out, b_out.reshape(1, D))
    return out


# speedup vs baseline: 1.1595x; 1.0371x over previous
import functools
import jax
import jax.numpy as jnp
from jax import lax
from jax.experimental import pallas as pl
from jax.experimental.pallas import tpu as pltpu
from jax.experimental.pallas import tpu_sc as plsc

N = 10000
D = 128
E = 320000
NC = 2
NS = 16
NW = NC * NS
EPT = E // NW
CH = 96
NFULL = EPT // CH
CT = EPT - NFULL * CH
CHD = 80
NCHD = EPT // CHD
ROWB = 1000
ZB = 40

_f32 = jnp.float32

_sc_mesh = plsc.VectorSubcoreMesh(core_axis_name="c", subcore_axis_name="s")


@functools.partial(
    pl.kernel,
    mesh=_sc_mesh,
    out_type=jax.ShapeDtypeStruct((NC * N,), _f32),
    scratch_types=[
        pltpu.VMEM((NCHD, CHD), jnp.int32),
        pltpu.VMEM((CHD,), _f32),
        pltpu.VMEM((ROWB,), _f32),
        pltpu.VMEM_SHARED((N,), _f32),
    ],
)
def _sc_deg(edges_hbm, ones_hbm, zeros_hbm, out_hbm, rowv, onesv, stg, degs):
    cid = lax.axis_index("c")
    sid = lax.axis_index("s")
    wid = cid * NS + sid

    pltpu.sync_copy(edges_hbm.at[wid], rowv)
    pltpu.sync_copy(ones_hbm, onesv)

    @pl.when(sid < N // ROWB)
    def _():
        pltpu.sync_copy(zeros_hbm, stg)
        pltpu.sync_copy(stg, degs.at[pl.ds(sid * ROWB, ROWB)])

    plsc.subcore_barrier()

    def body(g, carry):
        pltpu.sync_copy(onesv, degs.at[rowv.at[g]], add=True)
        return carry

    lax.fori_loop(0, NCHD, body, 0, unroll=False)

    plsc.subcore_barrier()

    @pl.when(sid < N // ROWB)
    def _():
        pltpu.sync_copy(degs.at[pl.ds(sid * ROWB, ROWB)], stg)
        pltpu.sync_copy(stg, out_hbm.at[pl.ds(cid * N + sid * ROWB, ROWB)])


@functools.partial(
    pl.kernel,
    mesh=_sc_mesh,
    out_type=jax.ShapeDtypeStruct((NC, N, D), _f32),
    scratch_types=[
        pltpu.VMEM((EPT,), jnp.int32),
        pltpu.VMEM((CH,), jnp.int32),
        pltpu.VMEM((CH,), jnp.int32),
        pltpu.VMEM((CH,), jnp.int32),
        pltpu.VMEM((CH,), jnp.int32),
        pltpu.VMEM((CT,), jnp.int32),
        pltpu.VMEM((CT,), jnp.int32),
        pltpu.VMEM((CH, D), _f32),
        pltpu.VMEM((CH, D), _f32),
        pltpu.VMEM_SHARED((N, D), _f32),
        pltpu.SemaphoreType.DMA,
        pltpu.SemaphoreType.DMA,
    ],
)
def _sc_spmm(epk_hbm, y_hbm, zeros_hbm, out_hbm, pk, rowca, colca,
             rowcb, colcb, rowt, colt, bufa, bufb, agg, sema, semb):
    cid = lax.axis_index("c")
    sid = lax.axis_index("s")
    wid = cid * NS + sid

    pltpu.sync_copy(epk_hbm.at[wid], pk)

    @pl.when(sid < N // ROWB)
    def _():
        pltpu.sync_copy(zeros_hbm, agg.at[pl.ds(sid * ROWB, ROWB)])

    plsc.subcore_barrier()

    def unpack(g, rowc, colc):
        for k in range(CH // 16):
            v = pk[pl.ds(g * CH + 16 * k, 16)]
            rowc[pl.ds(16 * k, 16)] = lax.shift_right_logical(v, 14)
            colc[pl.ds(16 * k, 16)] = lax.bitwise_and(v, (1 << 14) - 1)

    unpack(0, rowca, colca)
    pltpu.make_async_copy(y_hbm.at[colca], bufa, sema).start()

    def body(t, carry):
        ga = 2 * t
        unpack(ga + 1, rowcb, colcb)
        pltpu.make_async_copy(y_hbm.at[colcb], bufb, semb).start()
        pltpu.make_async_copy(y_hbm.at[colca], bufa, sema).wait()
        pltpu.sync_copy(bufa, agg.at[rowca], add=True)

        @pl.when(ga + 2 < NFULL)
        def _():
            unpack(ga + 2, rowca, colca)
            pltpu.make_async_copy(y_hbm.at[colca], bufa, sema).start()

        pltpu.make_async_copy(y_hbm.at[colcb], bufb, semb).wait()
        pltpu.sync_copy(bufb, agg.at[rowcb], add=True)
        return carry

    lax.fori_loop(0, NFULL // 2, body, 0, unroll=False)

    vt = pk[pl.ds(NFULL * CH, CT)]
    rowt[...] = lax.shift_right_logical(vt, 14)
    colt[...] = lax.bitwise_and(vt, (1 << 14) - 1)
    pltpu.sync_copy(y_hbm.at[colt], bufa.at[pl.ds(0, CT)])
    pltpu.sync_copy(bufa.at[pl.ds(0, CT)], agg.at[rowt], add=True)

    plsc.subcore_barrier()

    @pl.when(sid < N // ROWB)
    def _():
        pltpu.sync_copy(agg.at[pl.ds(sid * ROWB, ROWB)],
                        out_hbm.at[cid, pl.ds(sid * ROWB, ROWB)])


RB = 10000
GRID = N // RB


def _tc_prep_body(degp_ref, x_ref, w_ref, b_ref, dis_ref, h_ref, y_ref):
    deg = degp_ref[:, 0:1] + degp_ref[:, 1:2]
    dis = jnp.where(deg > 0.0,
                    lax.rsqrt(jnp.maximum(deg, 1e-12)), 0.0)
    h = lax.dot_general(x_ref[...], w_ref[...],
                        (((1,), (1,)), ((), ())),
                        preferred_element_type=_f32) + b_ref[...]
    dis_ref[...] = dis
    h_ref[...] = h
    y_ref[...] = h * dis


def _tc_prep(degp, x, w_in, b_in):
    return pl.pallas_call(
        _tc_prep_body,
        grid=(GRID,),
        in_specs=[
            pl.BlockSpec((RB, NC), lambda b: (b, 0)),
            pl.BlockSpec((RB, D), lambda b: (b, 0)),
            pl.BlockSpec((D, D), lambda b: (0, 0)),
            pl.BlockSpec((1, D), lambda b: (0, 0)),
        ],
        out_specs=[
            pl.BlockSpec((RB, 1), lambda b: (b, 0)),
            pl.BlockSpec((RB, D), lambda b: (b, 0)),
            pl.BlockSpec((RB, D), lambda b: (b, 0)),
        ],
        out_shape=[
            jax.ShapeDtypeStruct((N, 1), _f32),
            jax.ShapeDtypeStruct((N, D), _f32),
            jax.ShapeDtypeStruct((N, D), _f32),
        ],
    )(degp, x, w_in, b_in)


def _bn_from_acc(acc_ref):
    mean = acc_ref[0:1, :] / float(N)
    var = acc_ref[1:2, :] / float(N) - mean * mean
    return mean, lax.rsqrt(var + 1e-5)


def _layer_phase1(bm, part_ref, dis_ref, w_ref, b_ref, tbuf_ref, acc_ref):
    b = pl.program_id(0)
    a = (part_ref[0] + part_ref[1]) * dis_ref[...]
    t = lax.dot_general(a, w_ref[...], (((1,), (1,)), ((), ())),
                        preferred_element_type=_f32) + b_ref[...]
    tbuf_ref[bm] = t

    @pl.when(b == 0)
    def _():
        acc_ref[...] = jnp.zeros_like(acc_ref)

    acc_ref[0:1, :] += jnp.sum(t, axis=0, keepdims=True)
    acc_ref[1:2, :] += jnp.sum(t * t, axis=0, keepdims=True)


def _tc_layer_body(part_ref, dis_ref, w_ref, b_ref, h_ref, g_ref, be_ref,
                   hn_ref, y_ref, tbuf_ref, acc_ref):
    b = pl.program_id(0)
    bm = lax.rem(b, GRID)

    @pl.when(b < GRID)
    def _():
        _layer_phase1(bm, part_ref, dis_ref, w_ref, b_ref, tbuf_ref, acc_ref)

    @pl.when(b >= GRID)
    def _():
        mean, inv = _bn_from_acc(acc_ref)
        tn = (tbuf_ref[bm] - mean) * inv * g_ref[...] + be_ref[...]
        hn = jnp.maximum(tn, 0.0) + h_ref[...]
        hn_ref[...] = hn
        y_ref[...] = hn * dis_ref[...]


def _tc_layer(partial, dis, w, bvec, h, gamma, beta):
    return pl.pallas_call(
        _tc_layer_body,
        grid=(2 * GRID,),
        in_specs=[
            pl.BlockSpec((NC, RB, D), lambda b: (0, lax.min(b, GRID - 1), 0)),
            pl.BlockSpec((RB, 1), lambda b: (lax.rem(b, GRID), 0)),
            pl.BlockSpec((D, D), lambda b: (0, 0)),
            pl.BlockSpec((1, D), lambda b: (0, 0)),
            pl.BlockSpec((RB, D), lambda b: (lax.max(b - GRID, 0), 0)),
            pl.BlockSpec((1, D), lambda b: (0, 0)),
            pl.BlockSpec((1, D), lambda b: (0, 0)),
        ],
        out_specs=[
            pl.BlockSpec((RB, D), lambda b: (lax.rem(b, GRID), 0)),
            pl.BlockSpec((RB, D), lambda b: (lax.rem(b, GRID), 0)),
        ],
        out_shape=[
            jax.ShapeDtypeStruct((N, D), _f32),
            jax.ShapeDtypeStruct((N, D), _f32),
        ],
        scratch_shapes=[
            pltpu.VMEM((GRID, RB, D), _f32),
            pltpu.VMEM((2, D), _f32),
        ],
    )(partial, dis, w, bvec, h, gamma, beta)


def _tc_layer_out_body(part_ref, dis_ref, w_ref, b_ref, h_ref, g_ref, be_ref,
                       wo_ref, bo_ref, out_ref, tbuf_ref, acc_ref):
    b = pl.program_id(0)
    bm = lax.rem(b, GRID)

    @pl.when(b < GRID)
    def _():
        _layer_phase1(bm, part_ref, dis_ref, w_ref, b_ref, tbuf_ref, acc_ref)

    @pl.when(b >= GRID)
    def _():
        mean, inv = _bn_from_acc(acc_ref)
        tn = (tbuf_ref[bm] - mean) * inv * g_ref[...] + be_ref[...]
        hn = jnp.maximum(tn, 0.0) + h_ref[...]
        out_ref[...] = lax.dot_general(
            hn, wo_ref[...], (((1,), (1,)), ((), ())),
            preferred_element_type=_f32) + bo_ref[...]


def _tc_layer_out(partial, dis, w, bvec, h, gamma, beta, w_out, b_out):
    return pl.pallas_call(
        _tc_layer_out_body,
        grid=(2 * GRID,),
        in_specs=[
            pl.BlockSpec((NC, RB, D), lambda b: (0, lax.min(b, GRID - 1), 0)),
            pl.BlockSpec((RB, 1), lambda b: (lax.rem(b, GRID), 0)),
            pl.BlockSpec((D, D), lambda b: (0, 0)),
            pl.BlockSpec((1, D), lambda b: (0, 0)),
            pl.BlockSpec((RB, D), lambda b: (lax.max(b - GRID, 0), 0)),
            pl.BlockSpec((1, D), lambda b: (0, 0)),
            pl.BlockSpec((1, D), lambda b: (0, 0)),
            pl.BlockSpec((D, D), lambda b: (0, 0)),
            pl.BlockSpec((1, D), lambda b: (0, 0)),
        ],
        out_specs=pl.BlockSpec((RB, D), lambda b: (lax.rem(b, GRID), 0)),
        out_shape=jax.ShapeDtypeStruct((N, D), _f32),
        scratch_shapes=[
            pltpu.VMEM((GRID, RB, D), _f32),
            pltpu.VMEM((2, D), _f32),
        ],
    )(partial, dis, w, bvec, h, gamma, beta, w_out, b_out)


def kernel(x, edge_index, W_in, b_in, W1, b1, gamma1, beta1,
           W2, b2, gamma2, beta2, W_out, b_out):
    ei = edge_index.astype(jnp.int32)
    rows = ei[0].reshape(NW, NCHD, CHD)
    epacked = (ei[0] * (1 << 14) + ei[1]).reshape(NW, EPT)
    ones_e = jnp.ones((CHD,), _f32)
    zeros_n = jnp.zeros((ROWB,), _f32)
    zeros_zd = jnp.zeros((ROWB, D), _f32)

    degp = _sc_deg(rows, ones_e, zeros_n)
    dis, h, y = _tc_prep(degp.reshape(NC, N).T, x, W_in, b_in.reshape(1, D))

    part = _sc_spmm(epacked, y, zeros_zd)
    h, y = _tc_layer(part, dis, W1, b1.reshape(1, D), h,
                     gamma1.reshape(1, D), beta1.reshape(1, D))

    part = _sc_spmm(epacked, y, zeros_zd)
    out = _tc_layer_out(part, dis, W2, b2.reshape(1, D), h,
                        gamma2.reshape(1, D), beta2.reshape(1, D),
                        W_out, b_out.reshape(1, D))
    return out


# speedup vs baseline: 1.1883x; 1.0249x over previous
import functools
import jax
import jax.numpy as jnp
from jax import lax
from jax.experimental import pallas as pl
from jax.experimental.pallas import tpu as pltpu
from jax.experimental.pallas import tpu_sc as plsc

N = 10000
D = 128
E = 320000
NC = 2
NS = 16
NW = NC * NS
EPT = E // NW
CH = 112
NFULL = EPT // CH
CT = EPT - NFULL * CH
CHD = 80
NCHD = EPT // CHD
ROWB = 1000
ZB = 40

_f32 = jnp.float32

_sc_mesh = plsc.VectorSubcoreMesh(core_axis_name="c", subcore_axis_name="s")


@functools.partial(
    pl.kernel,
    mesh=_sc_mesh,
    out_type=jax.ShapeDtypeStruct((NC * N,), _f32),
    scratch_types=[
        pltpu.VMEM((NCHD, CHD), jnp.int32),
        pltpu.VMEM((CHD,), _f32),
        pltpu.VMEM((ROWB,), _f32),
        pltpu.VMEM_SHARED((N,), _f32),
    ],
)
def _sc_deg(edges_hbm, ones_hbm, zeros_hbm, out_hbm, rowv, onesv, stg, degs):
    cid = lax.axis_index("c")
    sid = lax.axis_index("s")
    wid = cid * NS + sid

    pltpu.sync_copy(edges_hbm.at[wid], rowv)
    pltpu.sync_copy(ones_hbm, onesv)

    @pl.when(sid < N // ROWB)
    def _():
        pltpu.sync_copy(zeros_hbm, stg)
        pltpu.sync_copy(stg, degs.at[pl.ds(sid * ROWB, ROWB)])

    plsc.subcore_barrier()

    def body(g, carry):
        pltpu.sync_copy(onesv, degs.at[rowv.at[g]], add=True)
        return carry

    lax.fori_loop(0, NCHD, body, 0, unroll=False)

    plsc.subcore_barrier()

    @pl.when(sid < N // ROWB)
    def _():
        pltpu.sync_copy(degs.at[pl.ds(sid * ROWB, ROWB)], stg)
        pltpu.sync_copy(stg, out_hbm.at[pl.ds(cid * N + sid * ROWB, ROWB)])


@functools.partial(
    pl.kernel,
    mesh=_sc_mesh,
    out_type=jax.ShapeDtypeStruct((NC, N, D), _f32),
    scratch_types=[
        pltpu.VMEM((EPT,), jnp.int32),
        pltpu.VMEM((CH,), jnp.int32),
        pltpu.VMEM((CH,), jnp.int32),
        pltpu.VMEM((CH,), jnp.int32),
        pltpu.VMEM((CH,), jnp.int32),
        pltpu.VMEM((CT,), jnp.int32),
        pltpu.VMEM((CT,), jnp.int32),
        pltpu.VMEM((CH, D), _f32),
        pltpu.VMEM((CH, D), _f32),
        pltpu.VMEM_SHARED((N, D), _f32),
        pltpu.SemaphoreType.DMA,
        pltpu.SemaphoreType.DMA,
    ],
)
def _sc_spmm(epk_hbm, y_hbm, zeros_hbm, out_hbm, pk, rowca, colca,
             rowcb, colcb, rowt, colt, bufa, bufb, agg, sema, semb):
    cid = lax.axis_index("c")
    sid = lax.axis_index("s")
    wid = cid * NS + sid

    pltpu.sync_copy(epk_hbm.at[wid], pk)

    @pl.when(sid < N // ROWB)
    def _():
        pltpu.sync_copy(zeros_hbm, agg.at[pl.ds(sid * ROWB, ROWB)])

    plsc.subcore_barrier()

    def unpack(g, rowc, colc):
        for k in range(CH // 16):
            v = pk[pl.ds(g * CH + 16 * k, 16)]
            rowc[pl.ds(16 * k, 16)] = lax.shift_right_logical(v, 14)
            colc[pl.ds(16 * k, 16)] = lax.bitwise_and(v, (1 << 14) - 1)

    unpack(0, rowca, colca)
    pltpu.make_async_copy(y_hbm.at[colca], bufa, sema).start()

    def body(t, carry):
        ga = 2 * t
        unpack(ga + 1, rowcb, colcb)
        pltpu.make_async_copy(y_hbm.at[colcb], bufb, semb).start()
        pltpu.make_async_copy(y_hbm.at[colca], bufa, sema).wait()
        pltpu.sync_copy(bufa, agg.at[rowca], add=True)

        @pl.when(ga + 2 < NFULL)
        def _():
            unpack(ga + 2, rowca, colca)
            pltpu.make_async_copy(y_hbm.at[colca], bufa, sema).start()

        pltpu.make_async_copy(y_hbm.at[colcb], bufb, semb).wait()
        pltpu.sync_copy(bufb, agg.at[rowcb], add=True)
        return carry

    lax.fori_loop(0, NFULL // 2, body, 0, unroll=False)

    pltpu.make_async_copy(y_hbm.at[colca], bufa, sema).wait()
    pltpu.sync_copy(bufa, agg.at[rowca], add=True)

    for k in range(CT // 16):
        vt = pk[pl.ds(NFULL * CH + 16 * k, 16)]
        rowt[pl.ds(16 * k, 16)] = lax.shift_right_logical(vt, 14)
        colt[pl.ds(16 * k, 16)] = lax.bitwise_and(vt, (1 << 14) - 1)
    pltpu.sync_copy(y_hbm.at[colt], bufa.at[pl.ds(0, CT)])
    pltpu.sync_copy(bufa.at[pl.ds(0, CT)], agg.at[rowt], add=True)

    plsc.subcore_barrier()

    @pl.when(sid < N // ROWB)
    def _():
        pltpu.sync_copy(agg.at[pl.ds(sid * ROWB, ROWB)],
                        out_hbm.at[cid, pl.ds(sid * ROWB, ROWB)])


RB = 10000
GRID = N // RB


def _tc_prep_body(degp_ref, x_ref, w_ref, b_ref, dis_ref, h_ref, y_ref):
    deg = degp_ref[:, 0:1] + degp_ref[:, 1:2]
    dis = jnp.where(deg > 0.0,
                    lax.rsqrt(jnp.maximum(deg, 1e-12)), 0.0)
    h = lax.dot_general(x_ref[...], w_ref[...],
                        (((1,), (1,)), ((), ())),
                        preferred_element_type=_f32) + b_ref[...]
    dis_ref[...] = dis
    h_ref[...] = h
    y_ref[...] = h * dis


def _tc_prep(degp, x, w_in, b_in):
    return pl.pallas_call(
        _tc_prep_body,
        grid=(GRID,),
        in_specs=[
            pl.BlockSpec((RB, NC), lambda b: (b, 0)),
            pl.BlockSpec((RB, D), lambda b: (b, 0)),
            pl.BlockSpec((D, D), lambda b: (0, 0)),
            pl.BlockSpec((1, D), lambda b: (0, 0)),
        ],
        out_specs=[
            pl.BlockSpec((RB, 1), lambda b: (b, 0)),
            pl.BlockSpec((RB, D), lambda b: (b, 0)),
            pl.BlockSpec((RB, D), lambda b: (b, 0)),
        ],
        out_shape=[
            jax.ShapeDtypeStruct((N, 1), _f32),
            jax.ShapeDtypeStruct((N, D), _f32),
            jax.ShapeDtypeStruct((N, D), _f32),
        ],
    )(degp, x, w_in, b_in)


def _bn_from_acc(acc_ref):
    mean = acc_ref[0:1, :] / float(N)
    var = acc_ref[1:2, :] / float(N) - mean * mean
    return mean, lax.rsqrt(var + 1e-5)


def _layer_phase1(bm, part_ref, dis_ref, w_ref, b_ref, tbuf_ref, acc_ref):
    b = pl.program_id(0)
    a = (part_ref[0] + part_ref[1]) * dis_ref[...]
    t = lax.dot_general(a, w_ref[...], (((1,), (1,)), ((), ())),
                        preferred_element_type=_f32) + b_ref[...]
    tbuf_ref[bm] = t

    @pl.when(b == 0)
    def _():
        acc_ref[...] = jnp.zeros_like(acc_ref)

    acc_ref[0:1, :] += jnp.sum(t, axis=0, keepdims=True)
    acc_ref[1:2, :] += jnp.sum(t * t, axis=0, keepdims=True)


def _tc_layer_body(part_ref, dis_ref, w_ref, b_ref, h_ref, g_ref, be_ref,
                   hn_ref, y_ref, tbuf_ref, acc_ref):
    b = pl.program_id(0)
    bm = lax.rem(b, GRID)

    @pl.when(b < GRID)
    def _():
        _layer_phase1(bm, part_ref, dis_ref, w_ref, b_ref, tbuf_ref, acc_ref)

    @pl.when(b >= GRID)
    def _():
        mean, inv = _bn_from_acc(acc_ref)
        tn = (tbuf_ref[bm] - mean) * inv * g_ref[...] + be_ref[...]
        hn = jnp.maximum(tn, 0.0) + h_ref[...]
        hn_ref[...] = hn
        y_ref[...] = hn * dis_ref[...]


def _tc_layer(partial, dis, w, bvec, h, gamma, beta):
    return pl.pallas_call(
        _tc_layer_body,
        grid=(2 * GRID,),
        in_specs=[
            pl.BlockSpec((NC, RB, D), lambda b: (0, lax.min(b, GRID - 1), 0)),
            pl.BlockSpec((RB, 1), lambda b: (lax.rem(b, GRID), 0)),
            pl.BlockSpec((D, D), lambda b: (0, 0)),
            pl.BlockSpec((1, D), lambda b: (0, 0)),
            pl.BlockSpec((RB, D), lambda b: (lax.max(b - GRID, 0), 0)),
            pl.BlockSpec((1, D), lambda b: (0, 0)),
            pl.BlockSpec((1, D), lambda b: (0, 0)),
        ],
        out_specs=[
            pl.BlockSpec((RB, D), lambda b: (lax.rem(b, GRID), 0)),
            pl.BlockSpec((RB, D), lambda b: (lax.rem(b, GRID), 0)),
        ],
        out_shape=[
            jax.ShapeDtypeStruct((N, D), _f32),
            jax.ShapeDtypeStruct((N, D), _f32),
        ],
        scratch_shapes=[
            pltpu.VMEM((GRID, RB, D), _f32),
            pltpu.VMEM((2, D), _f32),
        ],
    )(partial, dis, w, bvec, h, gamma, beta)


def _tc_layer_out_body(part_ref, dis_ref, w_ref, b_ref, h_ref, g_ref, be_ref,
                       wo_ref, bo_ref, out_ref, tbuf_ref, acc_ref):
    b = pl.program_id(0)
    bm = lax.rem(b, GRID)

    @pl.when(b < GRID)
    def _():
        _layer_phase1(bm, part_ref, dis_ref, w_ref, b_ref, tbuf_ref, acc_ref)

    @pl.when(b >= GRID)
    def _():
        mean, inv = _bn_from_acc(acc_ref)
        tn = (tbuf_ref[bm] - mean) * inv * g_ref[...] + be_ref[...]
        hn = jnp.maximum(tn, 0.0) + h_ref[...]
        out_ref[...] = lax.dot_general(
            hn, wo_ref[...], (((1,), (1,)), ((), ())),
            preferred_element_type=_f32) + bo_ref[...]


def _tc_layer_out(partial, dis, w, bvec, h, gamma, beta, w_out, b_out):
    return pl.pallas_call(
        _tc_layer_out_body,
        grid=(2 * GRID,),
        in_specs=[
            pl.BlockSpec((NC, RB, D), lambda b: (0, lax.min(b, GRID - 1), 0)),
            pl.BlockSpec((RB, 1), lambda b: (lax.rem(b, GRID), 0)),
            pl.BlockSpec((D, D), lambda b: (0, 0)),
            pl.BlockSpec((1, D), lambda b: (0, 0)),
            pl.BlockSpec((RB, D), lambda b: (lax.max(b - GRID, 0), 0)),
            pl.BlockSpec((1, D), lambda b: (0, 0)),
            pl.BlockSpec((1, D), lambda b: (0, 0)),
            pl.BlockSpec((D, D), lambda b: (0, 0)),
            pl.BlockSpec((1, D), lambda b: (0, 0)),
        ],
        out_specs=pl.BlockSpec((RB, D), lambda b: (lax.rem(b, GRID), 0)),
        out_shape=jax.ShapeDtypeStruct((N, D), _f32),
        scratch_shapes=[
            pltpu.VMEM((GRID, RB, D), _f32),
            pltpu.VMEM((2, D), _f32),
        ],
    )(partial, dis, w, bvec, h, gamma, beta, w_out, b_out)


def kernel(x, edge_index, W_in, b_in, W1, b1, gamma1, beta1,
           W2, b2, gamma2, beta2, W_out, b_out):
    ei = edge_index.astype(jnp.int32)
    rows = ei[0].reshape(NW, NCHD, CHD)
    epacked = (ei[0] * (1 << 14) + ei[1]).reshape(NW, EPT)
    ones_e = jnp.ones((CHD,), _f32)
    zeros_n = jnp.zeros((ROWB,), _f32)
    zeros_zd = jnp.zeros((ROWB, D), _f32)

    degp = _sc_deg(rows, ones_e, zeros_n)
    dis, h, y = _tc_prep(degp.reshape(NC, N).T, x, W_in, b_in.reshape(1, D))

    part = _sc_spmm(epacked, y, zeros_zd)
    h, y = _tc_layer(part, dis, W1, b1.reshape(1, D), h,
                     gamma1.reshape(1, D), beta1.reshape(1, D))

    part = _sc_spmm(epacked, y, zeros_zd)
    out = _tc_layer_out(part, dis, W2, b2.reshape(1, D), h,
                        gamma2.reshape(1, D), beta2.reshape(1, D),
                        W_out, b_out.reshape(1, D))
    return out
